# Initial kernel scaffold; baseline (speedup 1.0000x reference)
import jax
import jax.numpy as jnp
from jax.experimental import pallas as pl


def kernel(x, wkv_state, x_prev, memory_vec, rag_state, ssd_state, conv_state, norm_scale, norm_bias, W_in, conv_w, conv_b, W_dt, dt_bias, A_log, W_B, W_C, W_ssd_out, W_r, W_k, W_v, W_w, W_rwkv_out, W_skew, W_om_in, W_om_out, W_router, lora_A, lora_B, W_nov, b_nov, W_ragq, W_rago, W_mq, W_mp, W_mg, b_mg):
    raise NotImplementedError("write your pallas kernel here")


# speedup vs baseline: 35.7081x; 35.7081x over previous
import jax
import jax.numpy as jnp
from jax import lax
from jax.experimental import pallas as pl
from jax.experimental.pallas import tpu as pltpu

D = 1024
N = 64
P = 64
H = 16
E = 8
R = 8
OM = 32
MEMD = 384
KC = 4
Bz, Lz = 2, 1024

LT = 512
CS = 256
NCS = Lz // CS
CR = 64
NCR = Lz // CR

_F32 = jnp.float32


def _dot(a, b, ca, cb):
    return lax.dot_general(a, b, (((ca,), (cb,)), ((), ())),
                           preferred_element_type=_F32)


def _ln(x, s, b):
    m = jnp.mean(x, axis=-1, keepdims=True)
    v = jnp.mean((x - m) ** 2, axis=-1, keepdims=True)
    return (x - m) * lax.rsqrt(v + 1e-5) * s + b


def _silu(x):
    return x * (1.0 / (1.0 + jnp.exp(-x)))


def _sigmoid(x):
    return 1.0 / (1.0 + jnp.exp(-x))


def _k1_body(x_ref, xr_ref, xp_ref, ns_ref, nb_ref, wz_ref, wu_ref,
             wdbc_ref, wrkvw_ref, dtb_ref,
             z_ref, u_ref, bm_ref, cm_ref, dtp_ref, rr_ref, kk_ref,
             vv_ref, zww_ref, xlast_ref):
    j = pl.program_id(1)
    s = ns_ref[...]
    b = nb_ref[...]
    xn = _ln(x_ref[0], s, b)
    xnr = _ln(xr_ref[0], s, b)
    row0 = lax.broadcasted_iota(jnp.int32, (LT, 1), 0) == 0
    first = jnp.logical_and(row0, j == 0)
    xnr = jnp.where(first, xp_ref[0], xnr)
    mix = 0.5 * (xn + xnr)
    z_ref[0] = _dot(xn, wz_ref[...], 1, 1)
    u_ref[0] = _dot(xn, wu_ref[...], 1, 1)
    pbc = _dot(xn, wdbc_ref[...], 1, 1)
    bm_ref[0] = pbc[:, 0:64]
    cm_ref[0] = pbc[:, 64:128]
    dtp_ref[0] = jax.nn.softplus(pbc[:, 128:144] + dtb_ref[...])
    prk = _dot(mix, wrkvw_ref[...], 1, 1)
    rr_ref[0] = prk[:, 0:64]
    kk_ref[0] = prk[:, 64:128]
    vv_ref[0] = prk[:, 128:192]
    zww_ref[0] = prk[:, 192:256]

    @pl.when(j == (Lz // LT) - 1)
    def _():
        xlast_ref[0] = xn[LT - 1:LT, :]


def _run_k1(x, x_roll, x_prev, norm_scale, norm_bias, Wz, Wu, Wdbc, Wrkvw,
            dt_bias2):
    n_l = Lz // LT
    outs = (
        jax.ShapeDtypeStruct((Bz, Lz, D), _F32),
        jax.ShapeDtypeStruct((Bz, Lz, D), _F32),
        jax.ShapeDtypeStruct((Bz, Lz, N), _F32),
        jax.ShapeDtypeStruct((Bz, Lz, N), _F32),
        jax.ShapeDtypeStruct((Bz, Lz, H), _F32),
        jax.ShapeDtypeStruct((Bz, Lz, N), _F32),
        jax.ShapeDtypeStruct((Bz, Lz, N), _F32),
        jax.ShapeDtypeStruct((Bz, Lz, N), _F32),
        jax.ShapeDtypeStruct((Bz, Lz, N), _F32),
        jax.ShapeDtypeStruct((Bz, 1, D), _F32),
    )
    big = lambda b_, j_: (b_, j_, 0)
    w0 = lambda b_, j_: (0, 0)
    in_specs = [
        pl.BlockSpec((1, LT, D), big),
        pl.BlockSpec((1, LT, D), big),
        pl.BlockSpec((1, 1, D), lambda b_, j_: (b_, 0, 0)),
        pl.BlockSpec((D,), lambda b_, j_: (0,)),
        pl.BlockSpec((D,), lambda b_, j_: (0,)),
        pl.BlockSpec((D, D), w0),
        pl.BlockSpec((D, D), w0),
        pl.BlockSpec((256, D), w0),
        pl.BlockSpec((256, D), w0),
        pl.BlockSpec((1, H), w0),
    ]
    out_specs = (
        [pl.BlockSpec((1, LT, D), big)] * 2
        + [pl.BlockSpec((1, LT, N), big)] * 2
        + [pl.BlockSpec((1, LT, H), big)]
        + [pl.BlockSpec((1, LT, N), big)] * 4
        + [pl.BlockSpec((1, 1, D), lambda b_, j_: (b_, 0, 0))]
    )
    return pl.pallas_call(
        _k1_body,
        out_shape=outs,
        grid=(Bz, n_l),
        in_specs=in_specs,
        out_specs=out_specs,
        compiler_params=pltpu.CompilerParams(
            dimension_semantics=("parallel", "arbitrary"),
            vmem_limit_bytes=100 * 1024 * 1024,
        ),
        name="tars_k1_proj",
    )(x, x_roll, x_prev, norm_scale, norm_bias, Wz, Wu, Wdbc, Wrkvw,
      dt_bias2)


def _k2_body(u_ref, bm_ref, cm_ref, dtp_ref, rr_ref, kk_ref, vv_ref,
             zww_ref, stpad_ref, cwt_ref, cb_ref, alog_ref, ssd0_ref,
             wkv0_ref,
             ys_ref, yr_ref, ssdn_ref, wkvn_ref, convt_ref,
             ue, uc, sscr):
    ue[0:8, :] = stpad_ref[0]
    ue[8:8 + Lz, :] = u_ref[0]
    acc = cb_ref[...]
    for k in range(KC):
        acc = acc + ue[4 + k:4 + k + Lz, :] * cwt_ref[k:k + 1, :]
    uc[...] = _silu(acc)
    convt_ref[0, 0:4, :] = u_ref[0, Lz - 4:Lz, :]
    convt_ref[0, 4:8, :] = jnp.zeros((4, D), _F32)

    A2 = -jnp.exp(alog_ref[...])
    it = lax.broadcasted_iota(jnp.int32, (CS, CS), 0)
    js = lax.broadcasted_iota(jnp.int32, (CS, CS), 1)
    trilf = (it >= js).astype(_F32)
    neg = jnp.float32(-1e30)
    sscr[...] = ssd0_ref[0]
    for c in range(NCS):
        r0 = c * CS
        dtc = dtp_ref[0, r0:r0 + CS, :]
        bmc = bm_ref[0, r0:r0 + CS, :]
        cmc = cm_ref[0, r0:r0 + CS, :]
        cum = _dot(trilf, dtc, 1, 0)
        cumT = cum.T
        dtT = dtc.T
        tot = cum[CS - 1:CS, :]
        e0 = jnp.exp(cum * A2)
        w0 = jnp.exp((tot - cum) * A2) * dtc
        dvec = jnp.exp(tot * A2)
        cb = _dot(cmc, bmc, 1, 1)
        ys_parts = []
        s_new = []
        for h in range(H):
            ah = A2[:, h:h + 1]
            ccol = cum[:, h:h + 1]
            crow = cumT[h:h + 1, :]
            g = jnp.exp(jnp.where(it >= js, (ccol - crow) * ah, neg))
            g = g * dtT[h:h + 1, :]
            uch = uc[r0:r0 + CS, h * P:(h + 1) * P]
            sh = sscr[h]
            y = _dot(cb * g, uch, 1, 0)
            y = y + _dot(cmc, sh, 1, 1) * e0[:, h:h + 1]
            ys_parts.append(y)
            m = _dot(uch * w0[:, h:h + 1], bmc, 0, 0)
            s_new.append(sh * dvec[:, h:h + 1] + m)
        ys_ref[0, r0:r0 + CS, :] = jnp.concatenate(ys_parts, axis=1)
        sscr[...] = jnp.stack(s_new, axis=0)
    ssdn_ref[0] = sscr[...]

    logw = -jnp.exp(zww_ref[0])
    itr = lax.broadcasted_iota(jnp.int32, (CR, CR), 0)
    jsr = lax.broadcasted_iota(jnp.int32, (CR, CR), 1)
    trilr = (itr >= jsr).astype(_F32)
    mask3 = (lax.broadcasted_iota(jnp.int32, (CR, CR, 1), 0)
             >= lax.broadcasted_iota(jnp.int32, (CR, CR, 1), 1))
    swk = wkv0_ref[0]
    for c in range(NCR):
        r0 = c * CR
        lw = logw[r0:r0 + CR, :]
        el = _dot(trilr, lw, 1, 0)
        rc = rr_ref[0, r0:r0 + CR, :]
        kc = kk_ref[0, r0:r0 + CR, :]
        vc = vv_ref[0, r0:r0 + CR, :]
        kr = _dot(rc, kc, 1, 1)
        e0 = jnp.exp(el)
        diff = el[:, None, :] - el[None, :, :]
        pd = jnp.exp(jnp.where(mask3, diff, neg))
        x3 = pd * kr[:, :, None] * vc[None, :, :]
        y = jnp.sum(x3, axis=1)
        y = y + _dot(rc, swk, 1, 1) * e0
        yr_ref[0, r0:r0 + CR, :] = y
        wts = jnp.exp(el[CR - 1:CR, :] - el)
        mc = _dot(vc * wts, kc, 0, 0)
        dcol = e0[CR - 1:CR, :].T
        swk = swk * dcol + mc
    wkvn_ref[0] = swk


def _run_k2(u, Bm, Cm, dtp, rr, kk, vv, zww, stpad, cwt, conv_b2, alog2,
            ssd_state, wkv_state):
    outs = (
        jax.ShapeDtypeStruct((Bz, Lz, D), _F32),
        jax.ShapeDtypeStruct((Bz, Lz, N), _F32),
        jax.ShapeDtypeStruct((Bz, H, P, N), _F32),
        jax.ShapeDtypeStruct((Bz, N, N), _F32),
        jax.ShapeDtypeStruct((Bz, 8, D), _F32),
    )
    pb = lambda b_: (b_, 0, 0)
    w0 = lambda b_: (0, 0)
    in_specs = [
        pl.BlockSpec((1, Lz, D), pb),
        pl.BlockSpec((1, Lz, N), pb),
        pl.BlockSpec((1, Lz, N), pb),
        pl.BlockSpec((1, Lz, H), pb),
        pl.BlockSpec((1, Lz, N), pb),
        pl.BlockSpec((1, Lz, N), pb),
        pl.BlockSpec((1, Lz, N), pb),
        pl.BlockSpec((1, Lz, N), pb),
        pl.BlockSpec((1, 8, D), pb),
        pl.BlockSpec((8, D), w0),
        pl.BlockSpec((1, D), w0),
        pl.BlockSpec((1, H), w0),
        pl.BlockSpec((1, H, P, N), lambda b_: (b_, 0, 0, 0)),
        pl.BlockSpec((1, N, N), pb),
    ]
    out_specs = (
        pl.BlockSpec((1, Lz, D), pb),
        pl.BlockSpec((1, Lz, N), pb),
        pl.BlockSpec((1, H, P, N), lambda b_: (b_, 0, 0, 0)),
        pl.BlockSpec((1, N, N), pb),
        pl.BlockSpec((1, 8, D), pb),
    )
    return pl.pallas_call(
        _k2_body,
        out_shape=outs,
        grid=(Bz,),
        in_specs=in_specs,
        out_specs=list(out_specs),
        scratch_shapes=[
            pltpu.VMEM((Lz + 8, D), _F32),
            pltpu.VMEM((Lz, D), _F32),
            pltpu.VMEM((H, P, N), _F32),
        ],
        compiler_params=pltpu.CompilerParams(
            dimension_semantics=("parallel",),
            vmem_limit_bytes=100 * 1024 * 1024,
        ),
        name="tars_k2_scans",
    )(u, Bm, Cm, dtp, rr, kk, vv, zww, stpad, cwt, conv_b2, alog2,
      ssd_state, wkv_state)


def _k3a_body(x_ref, z_ref, ys_ref, yr_ref, rag_ref, wssd_ref, wrwk_ref,
              wragq_ref, wrago_ref, wsk_ref, womin_ref, womout_ref,
              x1_ref):
    y_ssd = _dot(ys_ref[0], wssd_ref[...], 1, 1)
    y_r = _dot(yr_ref[0], wrwk_ref[...], 1, 1)
    x1 = x_ref[0] + _silu(z_ref[0]) * y_ssd + y_r
    xm = jnp.sum(x1, axis=0, keepdims=True) * (1.0 / Lz)
    q = _dot(xm, wragq_ref[...], 1, 1)
    info = _dot(q, rag_ref[0], 1, 1)
    x1 = x1 + 0.1 * _dot(info, wrago_ref[...], 1, 1)
    wsk = wsk_ref[...]
    ask = 0.5 * (wsk - wsk.T)
    eye = (lax.broadcasted_iota(jnp.int32, (OM, OM), 0)
           == lax.broadcasted_iota(jnp.int32, (OM, OM), 1)).astype(_F32)
    m = eye - ask
    r1 = jnp.max(jnp.sum(jnp.abs(m), axis=0))
    rinf = jnp.max(jnp.sum(jnp.abs(m), axis=1))
    xinv = m.T * (1.0 / (r1 * rinf))
    for _ in range(20):
        xinv = _dot(xinv, 2.0 * eye - _dot(m, xinv, 1, 0), 1, 0)
    qc = _dot(xinv, eye + ask, 1, 0)
    t1 = _dot(x1, womin_ref[...], 1, 1)
    t2 = _dot(t1, qc, 1, 1)
    x1_ref[0] = x1 + _dot(t2, womout_ref[...], 1, 1)


def _run_k3a(x, z, ys, yr, rag_state, W_ssd_out, W_rwkv_out, W_ragq,
             W_rago, W_skew, W_om_in, W_om_out):
    pb = lambda b_: (b_, 0, 0)
    w0 = lambda b_: (0, 0)
    in_specs = [
        pl.BlockSpec((1, Lz, D), pb),
        pl.BlockSpec((1, Lz, D), pb),
        pl.BlockSpec((1, Lz, D), pb),
        pl.BlockSpec((1, Lz, N), pb),
        pl.BlockSpec((1, N, N), pb),
        pl.BlockSpec((D, D), w0),
        pl.BlockSpec((D, N), w0),
        pl.BlockSpec((N, D), w0),
        pl.BlockSpec((D, N), w0),
        pl.BlockSpec((OM, OM), w0),
        pl.BlockSpec((OM, D), w0),
        pl.BlockSpec((D, OM), w0),
    ]
    return pl.pallas_call(
        _k3a_body,
        out_shape=jax.ShapeDtypeStruct((Bz, Lz, D), _F32),
        grid=(Bz,),
        in_specs=in_specs,
        out_specs=pl.BlockSpec((1, Lz, D), pb),
        compiler_params=pltpu.CompilerParams(
            dimension_semantics=("parallel",),
            vmem_limit_bytes=100 * 1024 * 1024,
        ),
        name="tars_k3a_fuse",
    )(x, z, ys, yr, rag_state, W_ssd_out, W_rwkv_out, W_ragq, W_rago,
      W_skew, W_om_in, W_om_out)


def _k3b_body(x_ref, x1_ref, mem_ref, wrt_ref, la_ref, lb_ref, wnova_ref,
              wnovb_ref, bnov_ref, wmq_ref, wmp_ref, wmga_ref, wmgb_ref,
              bmg_ref,
              x4_ref, auxb_ref):
    x1 = x1_ref[0]
    logits = _dot(x1, wrt_ref[...], 1, 1)
    mx = jnp.max(logits, axis=-1, keepdims=True)
    ex = jnp.exp(logits - mx)
    probs = ex / jnp.sum(ex, axis=-1, keepdims=True)
    io = lax.broadcasted_iota(jnp.int32, (Lz, E), 1)
    m1 = jnp.max(probs, axis=-1, keepdims=True)
    i1 = jnp.min(jnp.where(probs == m1, io, E), axis=-1, keepdims=True)
    mask1 = io == i1
    p2 = jnp.where(mask1, -1.0, probs)
    m2 = jnp.max(p2, axis=-1, keepdims=True)
    i2 = jnp.min(jnp.where(p2 == m2, io, E), axis=-1, keepdims=True)
    maskf = jnp.logical_or(mask1, io == i2).astype(_F32)
    gate = probs * maskf
    gate = gate / (jnp.sum(gate, axis=-1, keepdims=True) + 1e-9)
    acc = jnp.zeros((Lz, D), _F32)
    for e in range(E):
        d_e = _dot(x1, la_ref[e], 1, 0)
        acc = acc + _dot(d_e * gate[:, e:e + 1], lb_ref[e], 1, 0)
    x2 = x1 + acc
    psum = jnp.sum(probs, axis=0, keepdims=True)
    msum = jnp.sum(maskf, axis=0, keepdims=True)
    auxb_ref[0] = jnp.concatenate(
        [psum, msum, jnp.zeros((6, E), _F32)], axis=0)
    x = x_ref[0]
    h_old = jnp.sum(x, axis=0, keepdims=True) * (1.0 / Lz)
    h_new = jnp.sum(x2, axis=0, keepdims=True) * (1.0 / Lz)
    nov = _sigmoid(_dot(h_old, wnova_ref[...], 1, 1)
                   + _dot(h_new, wnovb_ref[...], 1, 1) + bnov_ref[0])
    x3 = nov * x2 + (1.0 - nov) * x
    h_post = nov * h_new + (1.0 - nov) * h_old
    hq = _dot(h_post, wmq_ref[...], 1, 1)
    mem = mem_ref[0]
    num = jnp.sum(hq * mem)
    den = jnp.sqrt(jnp.sum(hq * hq)) * jnp.sqrt(jnp.sum(mem * mem)) + 1e-8
    sim = num / den
    gm = _sigmoid(_dot(h_post, wmga_ref[...], 1, 1)
                  + _dot(mem, wmgb_ref[...], 1, 1) + bmg_ref[0])
    x4_ref[0] = x3 + (sim * gm) * _dot(mem, wmp_ref[...], 1, 1)


def _run_k3b(x, x1, mem2, W_router, lora_A, lora_B, Wnova, Wnovb, b_nov,
             W_mq, W_mp, Wmga, Wmgb, b_mg):
    pb = lambda b_: (b_, 0, 0)
    w0 = lambda b_: (0, 0)
    in_specs = [
        pl.BlockSpec((1, Lz, D), pb),
        pl.BlockSpec((1, Lz, D), pb),
        pl.BlockSpec((1, 1, MEMD), lambda b_: (b_, 0, 0)),
        pl.BlockSpec((E, D), w0),
        pl.BlockSpec((E, D, R), lambda b_: (0, 0, 0)),
        pl.BlockSpec((E, R, D), lambda b_: (0, 0, 0)),
        pl.BlockSpec((1, D), w0),
        pl.BlockSpec((1, D), w0),
        pl.BlockSpec(memory_space=pltpu.SMEM),
        pl.BlockSpec((MEMD, D), w0),
        pl.BlockSpec((D, MEMD), w0),
        pl.BlockSpec((1, D), w0),
        pl.BlockSpec((1, MEMD), w0),
        pl.BlockSpec(memory_space=pltpu.SMEM),
    ]
    outs = (
        jax.ShapeDtypeStruct((Bz, Lz, D), _F32),
        jax.ShapeDtypeStruct((Bz, 8, E), _F32),
    )
    out_specs = [
        pl.BlockSpec((1, Lz, D), pb),
        pl.BlockSpec((1, 8, E), pb),
    ]
    return pl.pallas_call(
        _k3b_body,
        out_shape=outs,
        grid=(Bz,),
        in_specs=in_specs,
        out_specs=out_specs,
        compiler_params=pltpu.CompilerParams(
            dimension_semantics=("parallel",),
            vmem_limit_bytes=100 * 1024 * 1024,
        ),
        name="tars_k3b_tail",
    )(x, x1, mem2, W_router, lora_A, lora_B, Wnova, Wnovb, b_nov,
      W_mq, W_mp, Wmga, Wmgb, b_mg)


def kernel(x, wkv_state, x_prev, memory_vec, rag_state, ssd_state,
           conv_state, norm_scale, norm_bias, W_in, conv_w, conv_b, W_dt,
           dt_bias, A_log, W_B, W_C, W_ssd_out, W_r, W_k, W_v, W_w,
           W_rwkv_out, W_skew, W_om_in, W_om_out, W_router, lora_A,
           lora_B, W_nov, b_nov, W_ragq, W_rago, W_mq, W_mp, W_mg, b_mg):
    f = _F32
    x_roll = jnp.concatenate([x[:, :1], x[:, :-1]], axis=1)
    Wz = W_in[:D]
    Wu = W_in[D:]
    Wdbc = jnp.concatenate(
        [W_B, W_C, W_dt, jnp.zeros((256 - 2 * N - H, D), f)], axis=0)
    Wrkvw = jnp.concatenate([W_r, W_k, W_v, W_w], axis=0)
    dt_bias2 = dt_bias.reshape(1, H)
    (z, u, Bm, Cm, dtp, rr, kk, vv, zww, xlast) = _run_k1(
        x, x_roll, x_prev, norm_scale, norm_bias, Wz, Wu, Wdbc, Wrkvw,
        dt_bias2)

    stpad = jnp.pad(conv_state.transpose(0, 2, 1), ((0, 0), (4, 0), (0, 0)))
    cwt = jnp.pad(conv_w.T, ((0, 8 - KC), (0, 0)))
    conv_b2 = conv_b.reshape(1, D)
    alog2 = A_log.reshape(1, H)
    ys, yr, ssd_new, wkv_new, convt = _run_k2(
        u, Bm, Cm, dtp, rr, kk, vv, zww, stpad, cwt, conv_b2, alog2,
        ssd_state, wkv_state)
    conv_new = convt[:, 0:4, :].transpose(0, 2, 1)

    x1 = _run_k3a(x, z, ys, yr, rag_state, W_ssd_out, W_rwkv_out, W_ragq,
                  W_rago, W_skew, W_om_in, W_om_out)

    Wnova = W_nov[:, :D]
    Wnovb = W_nov[:, D:]
    Wmga = W_mg[:, :D]
    Wmgb = W_mg[:, D:]
    x4, auxb = _run_k3b(x, x1, memory_vec.reshape(Bz, 1, MEMD), W_router,
                        lora_A, lora_B,
                        Wnova, Wnovb, b_nov, W_mq, W_mp, Wmga, Wmgb, b_mg)

    pmean = jnp.sum(auxb[:, 0, :], axis=0) * (1.0 / (Bz * Lz))
    mmean = jnp.sum(auxb[:, 1, :], axis=0) * (1.0 / (Bz * Lz))
    aux = E * jnp.sum(pmean * mmean)
    return x4, wkv_new, xlast, ssd_new, conv_new, aux


# speedup vs baseline: 38.9751x; 1.0915x over previous
import jax
import jax.numpy as jnp
from jax import lax
from jax.experimental import pallas as pl
from jax.experimental.pallas import tpu as pltpu

D = 1024
N = 64
P = 64
H = 16
E = 8
R = 8
OM = 32
MEMD = 384
KC = 4
Bz, Lz = 2, 1024

CS = 256
NCS = Lz // CS
CR = 64
NCR = Lz // CR

_F32 = jnp.float32
_VMEM = pl.BlockSpec(memory_space=pltpu.VMEM)
_SMEM = pl.BlockSpec(memory_space=pltpu.SMEM)


def _cp():
    return pltpu.CompilerParams(vmem_limit_bytes=100 * 1024 * 1024)


def _dot(a, b, ca, cb):
    return lax.dot_general(a, b, (((ca,), (cb,)), ((), ())),
                           preferred_element_type=_F32)


def _ln(x, s, b):
    m = jnp.mean(x, axis=-1, keepdims=True)
    v = jnp.mean((x - m) ** 2, axis=-1, keepdims=True)
    return (x - m) * lax.rsqrt(v + 1e-5) * s + b


def _silu(x):
    return x * (1.0 / (1.0 + jnp.exp(-x)))


def _sigmoid(x):
    return 1.0 / (1.0 + jnp.exp(-x))


def _k1_body(x_ref, xp_ref, ns_ref, nb_ref, wz_ref, wu_ref,
             wdbc_ref, wrkvw_ref, dtb_ref,
             z_ref, u_ref, bm_ref, cm_ref, dtp_ref, rr_ref, kk_ref,
             vv_ref, zww_ref, xlast_ref):
    s = ns_ref[...]
    b = nb_ref[...]
    for bi in range(Bz):
        xn = _ln(x_ref[bi], s, b)
        xs = jnp.concatenate([xp_ref[bi], xn[:Lz - 1]], axis=0)
        mix = 0.5 * (xn + xs)
        z_ref[bi] = _dot(xn, wz_ref[...], 1, 1)
        u_ref[bi] = _dot(xn, wu_ref[...], 1, 1)
        pbc = _dot(xn, wdbc_ref[...], 1, 1)
        bm_ref[bi] = pbc[:, 0:64]
        cm_ref[bi] = pbc[:, 64:128]
        dtp_ref[bi] = jax.nn.softplus(pbc[:, 128:144] + dtb_ref[...])
        prk = _dot(mix, wrkvw_ref[...], 1, 1)
        rr_ref[bi] = prk[:, 0:64]
        kk_ref[bi] = prk[:, 64:128]
        vv_ref[bi] = prk[:, 128:192]
        zww_ref[bi] = prk[:, 192:256]
        xlast_ref[bi] = xn[Lz - 1:Lz, :]


def _run_k1(x, x_prev, norm_scale, norm_bias, Wz, Wu, Wdbc, Wrkvw,
            dt_bias2):
    outs = (
        jax.ShapeDtypeStruct((Bz, Lz, D), _F32),
        jax.ShapeDtypeStruct((Bz, Lz, D), _F32),
        jax.ShapeDtypeStruct((Bz, Lz, N), _F32),
        jax.ShapeDtypeStruct((Bz, Lz, N), _F32),
        jax.ShapeDtypeStruct((Bz, Lz, H), _F32),
        jax.ShapeDtypeStruct((Bz, Lz, N), _F32),
        jax.ShapeDtypeStruct((Bz, Lz, N), _F32),
        jax.ShapeDtypeStruct((Bz, Lz, N), _F32),
        jax.ShapeDtypeStruct((Bz, Lz, N), _F32),
        jax.ShapeDtypeStruct((Bz, 1, D), _F32),
    )
    return pl.pallas_call(
        _k1_body,
        out_shape=outs,
        in_specs=[_VMEM] * 9,
        out_specs=tuple([_VMEM] * 10),
        compiler_params=_cp(),
        name="tars_k1_proj",
    )(x, x_prev, norm_scale, norm_bias, Wz, Wu, Wdbc, Wrkvw, dt_bias2)


def _k2_body(u_ref, bm_ref, cm_ref, dtp_ref, rr_ref, kk_ref, vv_ref,
             zww_ref, stpad_ref, cwt_ref, cb_ref, alog_ref, ssd0_ref,
             wkv0_ref,
             ys_ref, yr_ref, ssdn_ref, wkvn_ref, convt_ref,
             ue, uc, sscr):
    A2 = -jnp.exp(alog_ref[...])
    it = lax.broadcasted_iota(jnp.int32, (CS, CS), 0)
    js = lax.broadcasted_iota(jnp.int32, (CS, CS), 1)
    trilf = (it >= js).astype(_F32)
    itr = lax.broadcasted_iota(jnp.int32, (CR, CR), 0)
    jsr = lax.broadcasted_iota(jnp.int32, (CR, CR), 1)
    trilr = (itr >= jsr).astype(_F32)
    mask3 = (lax.broadcasted_iota(jnp.int32, (CR, CR, 1), 0)
             >= lax.broadcasted_iota(jnp.int32, (CR, CR, 1), 1))
    neg = jnp.float32(-1e30)
    for bi in range(Bz):
        ue[0:8, :] = stpad_ref[bi]
        ue[8:8 + Lz, :] = u_ref[bi]
        acc = cb_ref[...]
        for k in range(KC):
            acc = acc + ue[4 + k:4 + k + Lz, :] * cwt_ref[k:k + 1, :]
        uc[...] = _silu(acc)
        convt_ref[bi, 0:4, :] = u_ref[bi, Lz - 4:Lz, :]
        convt_ref[bi, 4:8, :] = jnp.zeros((4, D), _F32)

        sscr[...] = ssd0_ref[bi]
        for c in range(NCS):
            r0 = c * CS
            dtc = dtp_ref[bi, r0:r0 + CS, :]
            bmc = bm_ref[bi, r0:r0 + CS, :]
            cmc = cm_ref[bi, r0:r0 + CS, :]
            cum = _dot(trilf, dtc, 1, 0)
            cumT = cum.T
            dtT = dtc.T
            tot = cum[CS - 1:CS, :]
            e0 = jnp.exp(cum * A2)
            w0 = jnp.exp((tot - cum) * A2) * dtc
            dvec = jnp.exp(tot * A2)
            cb = _dot(cmc, bmc, 1, 1)
            ys_parts = []
            s_new = []
            for h in range(H):
                ah = A2[:, h:h + 1]
                ccol = cum[:, h:h + 1]
                crow = cumT[h:h + 1, :]
                g = jnp.exp(jnp.where(it >= js, (ccol - crow) * ah, neg))
                g = g * dtT[h:h + 1, :]
                uch = uc[r0:r0 + CS, h * P:(h + 1) * P]
                sh = sscr[h]
                y = _dot(cb * g, uch, 1, 0)
                y = y + _dot(cmc, sh, 1, 1) * e0[:, h:h + 1]
                ys_parts.append(y)
                m = _dot(uch * w0[:, h:h + 1], bmc, 0, 0)
                s_new.append(sh * dvec[:, h:h + 1] + m)
            ys_ref[bi, r0:r0 + CS, :] = jnp.concatenate(ys_parts, axis=1)
            sscr[...] = jnp.stack(s_new, axis=0)
        ssdn_ref[bi] = sscr[...]

        logw = -jnp.exp(zww_ref[bi])
        swk = wkv0_ref[bi]
        for c in range(NCR):
            r0 = c * CR
            lw = logw[r0:r0 + CR, :]
            el = _dot(trilr, lw, 1, 0)
            rc = rr_ref[bi, r0:r0 + CR, :]
            kc = kk_ref[bi, r0:r0 + CR, :]
            vc = vv_ref[bi, r0:r0 + CR, :]
            kr = _dot(rc, kc, 1, 1)
            e0 = jnp.exp(el)
            diff = el[:, None, :] - el[None, :, :]
            pd = jnp.exp(jnp.where(mask3, diff, neg))
            x3 = pd * kr[:, :, None] * vc[None, :, :]
            y = jnp.sum(x3, axis=1)
            y = y + _dot(rc, swk, 1, 1) * e0
            yr_ref[bi, r0:r0 + CR, :] = y
            wts = jnp.exp(el[CR - 1:CR, :] - el)
            mc = _dot(vc * wts, kc, 0, 0)
            dcol = e0[CR - 1:CR, :].T
            swk = swk * dcol + mc
        wkvn_ref[bi] = swk


def _run_k2(u, Bm, Cm, dtp, rr, kk, vv, zww, stpad, cwt, conv_b2, alog2,
            ssd_state, wkv_state):
    outs = (
        jax.ShapeDtypeStruct((Bz, Lz, D), _F32),
        jax.ShapeDtypeStruct((Bz, Lz, N), _F32),
        jax.ShapeDtypeStruct((Bz, H, P, N), _F32),
        jax.ShapeDtypeStruct((Bz, N, N), _F32),
        jax.ShapeDtypeStruct((Bz, 8, D), _F32),
    )
    return pl.pallas_call(
        _k2_body,
        out_shape=outs,
        in_specs=[_VMEM] * 14,
        out_specs=tuple([_VMEM] * 5),
        scratch_shapes=[
            pltpu.VMEM((Lz + 8, D), _F32),
            pltpu.VMEM((Lz, D), _F32),
            pltpu.VMEM((H, P, N), _F32),
        ],
        compiler_params=_cp(),
        name="tars_k2_scans",
    )(u, Bm, Cm, dtp, rr, kk, vv, zww, stpad, cwt, conv_b2, alog2,
      ssd_state, wkv_state)


def _k3a_body(x_ref, z_ref, ys_ref, yr_ref, rag_ref, wssd_ref, wrwk_ref,
              wragq_ref, wrago_ref, wsk_ref, womin_ref, womout_ref,
              x1_ref):
    wsk = wsk_ref[...]
    ask = 0.5 * (wsk - wsk.T)
    eye = (lax.broadcasted_iota(jnp.int32, (OM, OM), 0)
           == lax.broadcasted_iota(jnp.int32, (OM, OM), 1)).astype(_F32)
    m = eye - ask
    r1 = jnp.max(jnp.sum(jnp.abs(m), axis=0))
    rinf = jnp.max(jnp.sum(jnp.abs(m), axis=1))
    xinv = m.T * (1.0 / (r1 * rinf))
    for _ in range(20):
        xinv = _dot(xinv, 2.0 * eye - _dot(m, xinv, 1, 0), 1, 0)
    qc = _dot(xinv, eye + ask, 1, 0)
    for bi in range(Bz):
        y_ssd = _dot(ys_ref[bi], wssd_ref[...], 1, 1)
        y_r = _dot(yr_ref[bi], wrwk_ref[...], 1, 1)
        x1 = x_ref[bi] + _silu(z_ref[bi]) * y_ssd + y_r
        xm = jnp.sum(x1, axis=0, keepdims=True) * (1.0 / Lz)
        q = _dot(xm, wragq_ref[...], 1, 1)
        info = _dot(q, rag_ref[bi], 1, 1)
        x1 = x1 + 0.1 * _dot(info, wrago_ref[...], 1, 1)
        t1 = _dot(x1, womin_ref[...], 1, 1)
        t2 = _dot(t1, qc, 1, 1)
        x1_ref[bi] = x1 + _dot(t2, womout_ref[...], 1, 1)


def _run_k3a(x, z, ys, yr, rag_state, W_ssd_out, W_rwkv_out, W_ragq,
             W_rago, W_skew, W_om_in, W_om_out):
    return pl.pallas_call(
        _k3a_body,
        out_shape=jax.ShapeDtypeStruct((Bz, Lz, D), _F32),
        in_specs=[_VMEM] * 12,
        out_specs=_VMEM,
        compiler_params=_cp(),
        name="tars_k3a_fuse",
    )(x, z, ys, yr, rag_state, W_ssd_out, W_rwkv_out, W_ragq, W_rago,
      W_skew, W_om_in, W_om_out)


def _k3b_body(x_ref, x1_ref, mem_ref, wrt_ref, la_ref, lb_ref, wnova_ref,
              wnovb_ref, bnov_ref, wmq_ref, wmp_ref, wmga_ref, wmgb_ref,
              bmg_ref,
              x4_ref, auxb_ref):
    io = lax.broadcasted_iota(jnp.int32, (Lz, E), 1)
    for bi in range(Bz):
        x1 = x1_ref[bi]
        logits = _dot(x1, wrt_ref[...], 1, 1)
        mx = jnp.max(logits, axis=-1, keepdims=True)
        ex = jnp.exp(logits - mx)
        probs = ex / jnp.sum(ex, axis=-1, keepdims=True)
        m1 = jnp.max(probs, axis=-1, keepdims=True)
        i1 = jnp.min(jnp.where(probs == m1, io, E), axis=-1, keepdims=True)
        mask1 = io == i1
        p2 = jnp.where(mask1, -1.0, probs)
        m2 = jnp.max(p2, axis=-1, keepdims=True)
        i2 = jnp.min(jnp.where(p2 == m2, io, E), axis=-1, keepdims=True)
        maskf = jnp.logical_or(mask1, io == i2).astype(_F32)
        gate = probs * maskf
        gate = gate / (jnp.sum(gate, axis=-1, keepdims=True) + 1e-9)
        acc = jnp.zeros((Lz, D), _F32)
        for e in range(E):
            d_e = _dot(x1, la_ref[e], 1, 0)
            acc = acc + _dot(d_e * gate[:, e:e + 1], lb_ref[e], 1, 0)
        x2 = x1 + acc
        psum = jnp.sum(probs, axis=0, keepdims=True)
        msum = jnp.sum(maskf, axis=0, keepdims=True)
        auxb_ref[bi] = jnp.concatenate(
            [psum, msum, jnp.zeros((6, E), _F32)], axis=0)
        x = x_ref[bi]
        h_old = jnp.sum(x, axis=0, keepdims=True) * (1.0 / Lz)
        h_new = jnp.sum(x2, axis=0, keepdims=True) * (1.0 / Lz)
        nov = _sigmoid(_dot(h_old, wnova_ref[...], 1, 1)
                       + _dot(h_new, wnovb_ref[...], 1, 1) + bnov_ref[0])
        x3 = nov * x2 + (1.0 - nov) * x
        h_post = nov * h_new + (1.0 - nov) * h_old
        hq = _dot(h_post, wmq_ref[...], 1, 1)
        mem = mem_ref[bi]
        num = jnp.sum(hq * mem)
        den = (jnp.sqrt(jnp.sum(hq * hq)) * jnp.sqrt(jnp.sum(mem * mem))
               + 1e-8)
        sim = num / den
        gm = _sigmoid(_dot(h_post, wmga_ref[...], 1, 1)
                      + _dot(mem, wmgb_ref[...], 1, 1) + bmg_ref[0])
        x4_ref[bi] = x3 + (sim * gm) * _dot(mem, wmp_ref[...], 1, 1)


def _run_k3b(x, x1, mem2, W_router, lora_A, lora_B, Wnova, Wnovb, b_nov,
             W_mq, W_mp, Wmga, Wmgb, b_mg):
    in_specs = [_VMEM] * 8 + [_SMEM] + [_VMEM] * 4 + [_SMEM]
    outs = (
        jax.ShapeDtypeStruct((Bz, Lz, D), _F32),
        jax.ShapeDtypeStruct((Bz, 8, E), _F32),
    )
    return pl.pallas_call(
        _k3b_body,
        out_shape=outs,
        in_specs=in_specs,
        out_specs=(_VMEM, _VMEM),
        compiler_params=_cp(),
        name="tars_k3b_tail",
    )(x, x1, mem2, W_router, lora_A, lora_B, Wnova, Wnovb, b_nov,
      W_mq, W_mp, Wmga, Wmgb, b_mg)


def kernel(x, wkv_state, x_prev, memory_vec, rag_state, ssd_state,
           conv_state, norm_scale, norm_bias, W_in, conv_w, conv_b, W_dt,
           dt_bias, A_log, W_B, W_C, W_ssd_out, W_r, W_k, W_v, W_w,
           W_rwkv_out, W_skew, W_om_in, W_om_out, W_router, lora_A,
           lora_B, W_nov, b_nov, W_ragq, W_rago, W_mq, W_mp, W_mg, b_mg):
    f = _F32
    Wz = W_in[:D]
    Wu = W_in[D:]
    Wdbc = jnp.concatenate(
        [W_B, W_C, W_dt, jnp.zeros((256 - 2 * N - H, D), f)], axis=0)
    Wrkvw = jnp.concatenate([W_r, W_k, W_v, W_w], axis=0)
    dt_bias2 = dt_bias.reshape(1, H)
    (z, u, Bm, Cm, dtp, rr, kk, vv, zww, xlast) = _run_k1(
        x, x_prev, norm_scale, norm_bias, Wz, Wu, Wdbc, Wrkvw, dt_bias2)

    stpad = jnp.pad(conv_state.transpose(0, 2, 1), ((0, 0), (4, 0), (0, 0)))
    cwt = jnp.pad(conv_w.T, ((0, 8 - KC), (0, 0)))
    conv_b2 = conv_b.reshape(1, D)
    alog2 = A_log.reshape(1, H)
    ys, yr, ssd_new, wkv_new, convt = _run_k2(
        u, Bm, Cm, dtp, rr, kk, vv, zww, stpad, cwt, conv_b2, alog2,
        ssd_state, wkv_state)
    conv_new = convt[:, 0:4, :].transpose(0, 2, 1)

    x1 = _run_k3a(x, z, ys, yr, rag_state, W_ssd_out, W_rwkv_out, W_ragq,
                  W_rago, W_skew, W_om_in, W_om_out)

    Wnova = W_nov[:, :D]
    Wnovb = W_nov[:, D:]
    Wmga = W_mg[:, :D]
    Wmgb = W_mg[:, D:]
    x4, auxb = _run_k3b(x, x1, memory_vec.reshape(Bz, 1, MEMD), W_router,
                        lora_A, lora_B, Wnova, Wnovb, b_nov, W_mq, W_mp,
                        Wmga, Wmgb, b_mg)

    pmean = jnp.sum(auxb[:, 0, :], axis=0) * (1.0 / (Bz * Lz))
    mmean = jnp.sum(auxb[:, 1, :], axis=0) * (1.0 / (Bz * Lz))
    aux = E * jnp.sum(pmean * mmean)
    return x4, wkv_new, xlast, ssd_new, conv_new, aux


# speedup vs baseline: 40.8555x; 1.0482x over previous
import jax
import jax.numpy as jnp
from jax import lax
from jax.experimental import pallas as pl
from jax.experimental.pallas import tpu as pltpu

D = 1024
N = 64
P = 64
H = 16
E = 8
R = 8
OM = 32
MEMD = 384
KC = 4
Bz, Lz = 2, 1024

CS = 256
NCS = Lz // CS
CR = 64
NCR = Lz // CR

_F32 = jnp.float32
_VMEM = pl.BlockSpec(memory_space=pltpu.VMEM)
_SMEM = pl.BlockSpec(memory_space=pltpu.SMEM)


def _cp():
    return pltpu.CompilerParams(vmem_limit_bytes=100 * 1024 * 1024)


def _dot(a, b, ca, cb):
    return lax.dot_general(a, b, (((ca,), (cb,)), ((), ())),
                           preferred_element_type=_F32)


def _ln(x, s, b):
    m = jnp.mean(x, axis=-1, keepdims=True)
    v = jnp.mean((x - m) ** 2, axis=-1, keepdims=True)
    return (x - m) * lax.rsqrt(v + 1e-5) * s + b


def _silu(x):
    return x * (1.0 / (1.0 + jnp.exp(-x)))


def _sigmoid(x):
    return 1.0 / (1.0 + jnp.exp(-x))


def _ka_body(x_ref, xp_ref, ns_ref, nb_ref, wz_ref, wu_ref, wdbc_ref,
             wrkvw_ref, dtb_ref, stpad_ref, cwt_ref, cb_ref, alog_ref,
             ssd0_ref, wkv0_ref,
             z_ref, ys_ref, yr_ref, ssdn_ref, wkvn_ref, convt_ref,
             xlast_ref,
             ue, uc, sscr):
    s = ns_ref[...]
    bvec = nb_ref[...]
    A2 = -jnp.exp(alog_ref[...])
    it = lax.broadcasted_iota(jnp.int32, (CS, CS), 0)
    js = lax.broadcasted_iota(jnp.int32, (CS, CS), 1)
    itr = lax.broadcasted_iota(jnp.int32, (CR, CR), 0)
    jsr = lax.broadcasted_iota(jnp.int32, (CR, CR), 1)
    trilf = (it >= js).astype(_F32)
    trilr = (itr >= jsr).astype(_F32)
    mask3 = (lax.broadcasted_iota(jnp.int32, (CR, CR, 1), 0)
             >= lax.broadcasted_iota(jnp.int32, (CR, CR, 1), 1))
    neg = jnp.float32(-1e30)
    for bi in range(Bz):
        xn = _ln(x_ref[bi], s, bvec)
        xs = jnp.concatenate([xp_ref[bi], xn[:Lz - 1]], axis=0)
        mix = 0.5 * (xn + xs)
        z_ref[bi] = _dot(xn, wz_ref[...], 1, 1)
        u = _dot(xn, wu_ref[...], 1, 1)
        pbc = _dot(xn, wdbc_ref[...], 1, 1)
        bm = pbc[:, 0:64]
        cm = pbc[:, 64:128]
        dtp = jax.nn.softplus(pbc[:, 128:144] + dtb_ref[...])
        prk = _dot(mix, wrkvw_ref[...], 1, 1)
        xlast_ref[bi] = xn[Lz - 1:Lz, :]

        ue[0:8, :] = stpad_ref[bi]
        ue[8:8 + Lz, :] = u
        convt_ref[bi, 0:4, :] = u[Lz - 4:Lz, :]
        convt_ref[bi, 4:8, :] = jnp.zeros((4, D), _F32)
        acc = cb_ref[...]
        for k in range(KC):
            acc = acc + ue[4 + k:4 + k + Lz, :] * cwt_ref[k:k + 1, :]
        uc[...] = _silu(acc)

        sscr[...] = ssd0_ref[bi]
        for c in range(NCS):
            r0 = c * CS
            dtc = dtp[r0:r0 + CS, :]
            bmc = bm[r0:r0 + CS, :]
            cmc = cm[r0:r0 + CS, :]
            cum = _dot(trilf, dtc, 1, 0)
            cumT = cum.T
            dtT = dtc.T
            tot = cum[CS - 1:CS, :]
            e0 = jnp.exp(cum * A2)
            w0 = jnp.exp((tot - cum) * A2) * dtc
            dvec = jnp.exp(tot * A2)
            cb = _dot(cmc, bmc, 1, 1)
            ys_parts = []
            s_new = []
            for h in range(H):
                ah = A2[:, h:h + 1]
                ccol = cum[:, h:h + 1]
                crow = cumT[h:h + 1, :]
                g = jnp.exp(jnp.where(it >= js, (ccol - crow) * ah, neg))
                g = g * dtT[h:h + 1, :]
                uch = uc[r0:r0 + CS, h * P:(h + 1) * P]
                sh = sscr[h]
                y = _dot(cb * g, uch, 1, 0)
                y = y + _dot(cmc, sh, 1, 1) * e0[:, h:h + 1]
                ys_parts.append(y)
                m = _dot(uch * w0[:, h:h + 1], bmc, 0, 0)
                s_new.append(sh * dvec[:, h:h + 1] + m)
            ys_ref[bi, r0:r0 + CS, :] = jnp.concatenate(ys_parts, axis=1)
            sscr[...] = jnp.stack(s_new, axis=0)
        ssdn_ref[bi] = sscr[...]

        rfull = prk[:, 0:64]
        kfull = prk[:, 64:128]
        vfull = prk[:, 128:192]
        logw = -jnp.exp(prk[:, 192:256])
        swk = wkv0_ref[bi]
        for c in range(NCR):
            r0 = c * CR
            lw = logw[r0:r0 + CR, :]
            el = _dot(trilr, lw, 1, 0)
            rc = rfull[r0:r0 + CR, :]
            kc = kfull[r0:r0 + CR, :]
            vc = vfull[r0:r0 + CR, :]
            kr = _dot(rc, kc, 1, 1)
            e0 = jnp.exp(el)
            diff = el[:, None, :] - el[None, :, :]
            pd = jnp.exp(jnp.where(mask3, diff, neg))
            x3 = pd * kr[:, :, None] * vc[None, :, :]
            y = jnp.sum(x3, axis=1)
            y = y + _dot(rc, swk, 1, 1) * e0
            yr_ref[bi, r0:r0 + CR, :] = y
            wts = jnp.exp(el[CR - 1:CR, :] - el)
            mc = _dot(vc * wts, kc, 0, 0)
            dcol = e0[CR - 1:CR, :].T
            swk = swk * dcol + mc
        wkvn_ref[bi] = swk


def _run_ka(x, x_prev, norm_scale, norm_bias, Wz, Wu, Wdbc, Wrkvw,
            dt_bias2, stpad, cwt, conv_b2, alog2, ssd_state, wkv_state):
    outs = (
        jax.ShapeDtypeStruct((Bz, Lz, D), _F32),
        jax.ShapeDtypeStruct((Bz, Lz, D), _F32),
        jax.ShapeDtypeStruct((Bz, Lz, N), _F32),
        jax.ShapeDtypeStruct((Bz, H, P, N), _F32),
        jax.ShapeDtypeStruct((Bz, N, N), _F32),
        jax.ShapeDtypeStruct((Bz, 8, D), _F32),
        jax.ShapeDtypeStruct((Bz, 1, D), _F32),
    )
    return pl.pallas_call(
        _ka_body,
        out_shape=outs,
        in_specs=[_VMEM] * 15,
        out_specs=tuple([_VMEM] * 7),
        scratch_shapes=[
            pltpu.VMEM((Lz + 8, D), _F32),
            pltpu.VMEM((Lz, D), _F32),
            pltpu.VMEM((H, P, N), _F32),
        ],
        compiler_params=_cp(),
        name="tars_ka_proj_scans",
    )(x, x_prev, norm_scale, norm_bias, Wz, Wu, Wdbc, Wrkvw, dt_bias2,
      stpad, cwt, conv_b2, alog2, ssd_state, wkv_state)


def _k3a_body(x_ref, z_ref, ys_ref, yr_ref, rag_ref, wssd_ref, wrwk_ref,
              wragq_ref, wrago_ref, wsk_ref, womin_ref, womout_ref,
              x1_ref):
    wsk = wsk_ref[...]
    ask = 0.5 * (wsk - wsk.T)
    eye = (lax.broadcasted_iota(jnp.int32, (OM, OM), 0)
           == lax.broadcasted_iota(jnp.int32, (OM, OM), 1)).astype(_F32)
    m = eye - ask
    r1 = jnp.max(jnp.sum(jnp.abs(m), axis=0))
    rinf = jnp.max(jnp.sum(jnp.abs(m), axis=1))
    xinv = m.T * (1.0 / (r1 * rinf))
    for _ in range(20):
        xinv = _dot(xinv, 2.0 * eye - _dot(m, xinv, 1, 0), 1, 0)
    qc = _dot(xinv, eye + ask, 1, 0)
    for bi in range(Bz):
        y_ssd = _dot(ys_ref[bi], wssd_ref[...], 1, 1)
        y_r = _dot(yr_ref[bi], wrwk_ref[...], 1, 1)
        x1 = x_ref[bi] + _silu(z_ref[bi]) * y_ssd + y_r
        xm = jnp.sum(x1, axis=0, keepdims=True) * (1.0 / Lz)
        q = _dot(xm, wragq_ref[...], 1, 1)
        info = _dot(q, rag_ref[bi], 1, 1)
        x1 = x1 + 0.1 * _dot(info, wrago_ref[...], 1, 1)
        t1 = _dot(x1, womin_ref[...], 1, 1)
        t2 = _dot(t1, qc, 1, 1)
        x1_ref[bi] = x1 + _dot(t2, womout_ref[...], 1, 1)


def _run_k3a(x, z, ys, yr, rag_state, W_ssd_out, W_rwkv_out, W_ragq,
             W_rago, W_skew, W_om_in, W_om_out):
    return pl.pallas_call(
        _k3a_body,
        out_shape=jax.ShapeDtypeStruct((Bz, Lz, D), _F32),
        in_specs=[_VMEM] * 12,
        out_specs=_VMEM,
        compiler_params=_cp(),
        name="tars_k3a_fuse",
    )(x, z, ys, yr, rag_state, W_ssd_out, W_rwkv_out, W_ragq, W_rago,
      W_skew, W_om_in, W_om_out)


def _k3b_body(x_ref, x1_ref, mem_ref, wrt_ref, la_ref, lb_ref, wnova_ref,
              wnovb_ref, bnov_ref, wmq_ref, wmp_ref, wmga_ref, wmgb_ref,
              bmg_ref,
              x4_ref, auxb_ref):
    io = lax.broadcasted_iota(jnp.int32, (Lz, E), 1)
    for bi in range(Bz):
        x1 = x1_ref[bi]
        logits = _dot(x1, wrt_ref[...], 1, 1)
        mx = jnp.max(logits, axis=-1, keepdims=True)
        ex = jnp.exp(logits - mx)
        probs = ex / jnp.sum(ex, axis=-1, keepdims=True)
        m1 = jnp.max(probs, axis=-1, keepdims=True)
        i1 = jnp.min(jnp.where(probs == m1, io, E), axis=-1, keepdims=True)
        mask1 = io == i1
        p2 = jnp.where(mask1, -1.0, probs)
        m2 = jnp.max(p2, axis=-1, keepdims=True)
        i2 = jnp.min(jnp.where(p2 == m2, io, E), axis=-1, keepdims=True)
        maskf = jnp.logical_or(mask1, io == i2).astype(_F32)
        gate = probs * maskf
        gate = gate / (jnp.sum(gate, axis=-1, keepdims=True) + 1e-9)
        acc = jnp.zeros((Lz, D), _F32)
        for e in range(E):
            d_e = _dot(x1, la_ref[e], 1, 0)
            acc = acc + _dot(d_e * gate[:, e:e + 1], lb_ref[e], 1, 0)
        x2 = x1 + acc
        psum = jnp.sum(probs, axis=0, keepdims=True)
        msum = jnp.sum(maskf, axis=0, keepdims=True)
        auxb_ref[bi] = jnp.concatenate(
            [psum, msum, jnp.zeros((6, E), _F32)], axis=0)
        x = x_ref[bi]
        h_old = jnp.sum(x, axis=0, keepdims=True) * (1.0 / Lz)
        h_new = jnp.sum(x2, axis=0, keepdims=True) * (1.0 / Lz)
        nov = _sigmoid(_dot(h_old, wnova_ref[...], 1, 1)
                       + _dot(h_new, wnovb_ref[...], 1, 1) + bnov_ref[0])
        x3 = nov * x2 + (1.0 - nov) * x
        h_post = nov * h_new + (1.0 - nov) * h_old
        hq = _dot(h_post, wmq_ref[...], 1, 1)
        mem = mem_ref[bi]
        num = jnp.sum(hq * mem)
        den = (jnp.sqrt(jnp.sum(hq * hq)) * jnp.sqrt(jnp.sum(mem * mem))
               + 1e-8)
        sim = num / den
        gm = _sigmoid(_dot(h_post, wmga_ref[...], 1, 1)
                      + _dot(mem, wmgb_ref[...], 1, 1) + bmg_ref[0])
        x4_ref[bi] = x3 + (sim * gm) * _dot(mem, wmp_ref[...], 1, 1)


def _run_k3b(x, x1, mem2, W_router, lora_A, lora_B, Wnova, Wnovb, b_nov,
             W_mq, W_mp, Wmga, Wmgb, b_mg):
    in_specs = [_VMEM] * 8 + [_SMEM] + [_VMEM] * 4 + [_SMEM]
    outs = (
        jax.ShapeDtypeStruct((Bz, Lz, D), _F32),
        jax.ShapeDtypeStruct((Bz, 8, E), _F32),
    )
    return pl.pallas_call(
        _k3b_body,
        out_shape=outs,
        in_specs=in_specs,
        out_specs=(_VMEM, _VMEM),
        compiler_params=_cp(),
        name="tars_k3b_tail",
    )(x, x1, mem2, W_router, lora_A, lora_B, Wnova, Wnovb, b_nov,
      W_mq, W_mp, Wmga, Wmgb, b_mg)


def kernel(x, wkv_state, x_prev, memory_vec, rag_state, ssd_state,
           conv_state, norm_scale, norm_bias, W_in, conv_w, conv_b, W_dt,
           dt_bias, A_log, W_B, W_C, W_ssd_out, W_r, W_k, W_v, W_w,
           W_rwkv_out, W_skew, W_om_in, W_om_out, W_router, lora_A,
           lora_B, W_nov, b_nov, W_ragq, W_rago, W_mq, W_mp, W_mg, b_mg):
    f = _F32
    Wz = W_in[:D]
    Wu = W_in[D:]
    Wdbc = jnp.concatenate(
        [W_B, W_C, W_dt, jnp.zeros((256 - 2 * N - H, D), f)], axis=0)
    Wrkvw = jnp.concatenate([W_r, W_k, W_v, W_w], axis=0)
    dt_bias2 = dt_bias.reshape(1, H)
    stpad = jnp.pad(conv_state.transpose(0, 2, 1), ((0, 0), (4, 0), (0, 0)))
    cwt = jnp.pad(conv_w.T, ((0, 8 - KC), (0, 0)))
    conv_b2 = conv_b.reshape(1, D)
    alog2 = A_log.reshape(1, H)
    (z, ys, yr, ssd_new, wkv_new, convt, xlast) = _run_ka(
        x, x_prev, norm_scale, norm_bias, Wz, Wu, Wdbc, Wrkvw, dt_bias2,
        stpad, cwt, conv_b2, alog2, ssd_state, wkv_state)
    conv_new = convt[:, 0:4, :].transpose(0, 2, 1)

    x1 = _run_k3a(x, z, ys, yr, rag_state, W_ssd_out, W_rwkv_out, W_ragq,
                  W_rago, W_skew, W_om_in, W_om_out)

    Wnova = W_nov[:, :D]
    Wnovb = W_nov[:, D:]
    Wmga = W_mg[:, :D]
    Wmgb = W_mg[:, D:]
    x4, auxb = _run_k3b(x, x1, memory_vec.reshape(Bz, 1, MEMD), W_router,
                        lora_A, lora_B, Wnova, Wnovb, b_nov, W_mq, W_mp,
                        Wmga, Wmgb, b_mg)

    pmean = jnp.sum(auxb[:, 0, :], axis=0) * (1.0 / (Bz * Lz))
    mmean = jnp.sum(auxb[:, 1, :], axis=0) * (1.0 / (Bz * Lz))
    aux = E * jnp.sum(pmean * mmean)
    return x4, wkv_new, xlast, ssd_new, conv_new, aux


# speedup vs baseline: 45.1913x; 1.1061x over previous
import jax
import jax.numpy as jnp
from jax import lax
from jax.experimental import pallas as pl
from jax.experimental.pallas import tpu as pltpu

D = 1024
N = 64
P = 64
H = 16
E = 8
R = 8
OM = 32
MEMD = 384
KC = 4
Bz, Lz = 2, 1024

CS = 256
NCS = Lz // CS
CR = 64
NCR = Lz // CR

_F32 = jnp.float32
_VMEM = pl.BlockSpec(memory_space=pltpu.VMEM)
_SMEM = pl.BlockSpec(memory_space=pltpu.SMEM)


def _cp():
    return pltpu.CompilerParams(vmem_limit_bytes=100 * 1024 * 1024)


def _dot(a, b, ca, cb):
    return lax.dot_general(a, b, (((ca,), (cb,)), ((), ())),
                           preferred_element_type=_F32)


def _ln(x, s, b):
    m = jnp.mean(x, axis=-1, keepdims=True)
    v = jnp.mean((x - m) ** 2, axis=-1, keepdims=True)
    return (x - m) * lax.rsqrt(v + 1e-5) * s + b


def _silu(x):
    return x * (1.0 / (1.0 + jnp.exp(-x)))


def _sigmoid(x):
    return 1.0 / (1.0 + jnp.exp(-x))


def _ka_body(x_ref, xp_ref, ns_ref, nb_ref, wz_ref, wu_ref, wdbc_ref,
             wrkvw_ref, dtb_ref, stpad_ref, cwt_ref, cb_ref, alog_ref,
             ssd0_ref, wkv0_ref, wssd_ref, wrwk_ref,
             x1_ref, ssdn_ref, wkvn_ref, convt_ref, xlast_ref,
             ue, uc, zbuf, ysbuf, sscr):
    s = ns_ref[...]
    bvec = nb_ref[...]
    A2 = -jnp.exp(alog_ref[...])
    it = lax.broadcasted_iota(jnp.int32, (CS, CS), 0)
    js = lax.broadcasted_iota(jnp.int32, (CS, CS), 1)
    itr = lax.broadcasted_iota(jnp.int32, (CR, CR), 0)
    jsr = lax.broadcasted_iota(jnp.int32, (CR, CR), 1)
    trilf = (it >= js).astype(_F32)
    trilr = (itr >= jsr).astype(_F32)
    mask3 = (lax.broadcasted_iota(jnp.int32, (CR, CR, 1), 0)
             >= lax.broadcasted_iota(jnp.int32, (CR, CR, 1), 1))
    neg = jnp.float32(-1e30)
    for bi in range(Bz):
        xn = _ln(x_ref[bi], s, bvec)
        xs = jnp.concatenate([xp_ref[bi], xn[:Lz - 1]], axis=0)
        mix = 0.5 * (xn + xs)
        zbuf[...] = _dot(xn, wz_ref[...], 1, 1)
        u = _dot(xn, wu_ref[...], 1, 1)
        pbc = _dot(xn, wdbc_ref[...], 1, 1)
        bm = pbc[:, 0:64]
        cm = pbc[:, 64:128]
        dtp = jax.nn.softplus(pbc[:, 128:144] + dtb_ref[...])
        prk = _dot(mix, wrkvw_ref[...], 1, 1)
        xlast_ref[bi] = xn[Lz - 1:Lz, :]

        ue[0:8, :] = stpad_ref[bi]
        ue[8:8 + Lz, :] = u
        convt_ref[bi, 0:4, :] = u[Lz - 4:Lz, :]
        convt_ref[bi, 4:8, :] = jnp.zeros((4, D), _F32)
        acc = cb_ref[...]
        for k in range(KC):
            acc = acc + ue[4 + k:4 + k + Lz, :] * cwt_ref[k:k + 1, :]
        uc[...] = _silu(acc)

        sscr[...] = ssd0_ref[bi]
        for c in range(NCS):
            r0 = c * CS
            dtc = dtp[r0:r0 + CS, :]
            bmc = bm[r0:r0 + CS, :]
            cmc = cm[r0:r0 + CS, :]
            cum = _dot(trilf, dtc, 1, 0)
            cumT = cum.T
            dtT = dtc.T
            tot = cum[CS - 1:CS, :]
            e0 = jnp.exp(cum * A2)
            w0 = jnp.exp((tot - cum) * A2) * dtc
            dvec = jnp.exp(tot * A2)
            cb = _dot(cmc, bmc, 1, 1)
            ys_parts = []
            s_new = []
            for h in range(H):
                ah = A2[:, h:h + 1]
                ccol = cum[:, h:h + 1]
                crow = cumT[h:h + 1, :]
                g = jnp.exp(jnp.where(it >= js, (ccol - crow) * ah, neg))
                g = g * dtT[h:h + 1, :]
                uch = uc[r0:r0 + CS, h * P:(h + 1) * P]
                sh = sscr[h]
                y = _dot(cb * g, uch, 1, 0)
                y = y + _dot(cmc, sh, 1, 1) * e0[:, h:h + 1]
                ys_parts.append(y)
                m = _dot(uch * w0[:, h:h + 1], bmc, 0, 0)
                s_new.append(sh * dvec[:, h:h + 1] + m)
            ysbuf[r0:r0 + CS, :] = jnp.concatenate(ys_parts, axis=1)
            sscr[...] = jnp.stack(s_new, axis=0)
        ssdn_ref[bi] = sscr[...]

        rfull = prk[:, 0:64]
        kfull = prk[:, 64:128]
        vfull = prk[:, 128:192]
        logw = -jnp.exp(prk[:, 192:256])
        swk = wkv0_ref[bi]
        yr_parts = []
        for c in range(NCR):
            r0 = c * CR
            lw = logw[r0:r0 + CR, :]
            el = _dot(trilr, lw, 1, 0)
            rc = rfull[r0:r0 + CR, :]
            kc = kfull[r0:r0 + CR, :]
            vc = vfull[r0:r0 + CR, :]
            kr = _dot(rc, kc, 1, 1)
            e0 = jnp.exp(el)
            diff = el[:, None, :] - el[None, :, :]
            pd = jnp.exp(jnp.where(mask3, diff, neg))
            x3 = pd * kr[:, :, None] * vc[None, :, :]
            y = jnp.sum(x3, axis=1)
            yr_parts.append(y + _dot(rc, swk, 1, 1) * e0)
            wts = jnp.exp(el[CR - 1:CR, :] - el)
            mc = _dot(vc * wts, kc, 0, 0)
            dcol = e0[CR - 1:CR, :].T
            swk = swk * dcol + mc
        wkvn_ref[bi] = swk
        yrb = jnp.concatenate(yr_parts, axis=0)

        y_ssd = _dot(ysbuf[...], wssd_ref[...], 1, 1)
        y_r = _dot(yrb, wrwk_ref[...], 1, 1)
        x1_ref[bi] = x_ref[bi] + _silu(zbuf[...]) * y_ssd + y_r


def _run_ka(x, x_prev, norm_scale, norm_bias, Wz, Wu, Wdbc, Wrkvw,
            dt_bias2, stpad, cwt, conv_b2, alog2, ssd_state, wkv_state,
            W_ssd_out, W_rwkv_out):
    outs = (
        jax.ShapeDtypeStruct((Bz, Lz, D), _F32),
        jax.ShapeDtypeStruct((Bz, H, P, N), _F32),
        jax.ShapeDtypeStruct((Bz, N, N), _F32),
        jax.ShapeDtypeStruct((Bz, 8, D), _F32),
        jax.ShapeDtypeStruct((Bz, 1, D), _F32),
    )
    return pl.pallas_call(
        _ka_body,
        out_shape=outs,
        in_specs=[_VMEM] * 17,
        out_specs=tuple([_VMEM] * 5),
        scratch_shapes=[
            pltpu.VMEM((Lz + 8, D), _F32),
            pltpu.VMEM((Lz, D), _F32),
            pltpu.VMEM((Lz, D), _F32),
            pltpu.VMEM((Lz, D), _F32),
            pltpu.VMEM((H, P, N), _F32),
        ],
        compiler_params=_cp(),
        name="tars_ka_core",
    )(x, x_prev, norm_scale, norm_bias, Wz, Wu, Wdbc, Wrkvw, dt_bias2,
      stpad, cwt, conv_b2, alog2, ssd_state, wkv_state, W_ssd_out,
      W_rwkv_out)


def _kb_body(x_ref, x1_ref, rag_ref, mem_ref, wragq_ref, wrago_ref,
             wsk_ref, womin_ref, womout_ref, wrt_ref, la_ref, lb_ref,
             wnova_ref, wnovb_ref, bnov_ref, wmq_ref, wmp_ref, wmga_ref,
             wmgb_ref, bmg_ref,
             x4_ref, auxb_ref):
    wsk = wsk_ref[...]
    ask = 0.5 * (wsk - wsk.T)
    eye = (lax.broadcasted_iota(jnp.int32, (OM, OM), 0)
           == lax.broadcasted_iota(jnp.int32, (OM, OM), 1)).astype(_F32)
    m = eye - ask
    r1 = jnp.max(jnp.sum(jnp.abs(m), axis=0))
    rinf = jnp.max(jnp.sum(jnp.abs(m), axis=1))
    xinv = m.T * (1.0 / (r1 * rinf))
    for _ in range(20):
        xinv = _dot(xinv, 2.0 * eye - _dot(m, xinv, 1, 0), 1, 0)
    qc = _dot(xinv, eye + ask, 1, 0)
    io = lax.broadcasted_iota(jnp.int32, (Lz, E), 1)
    for bi in range(Bz):
        x1 = x1_ref[bi]
        xm = jnp.sum(x1, axis=0, keepdims=True) * (1.0 / Lz)
        q = _dot(xm, wragq_ref[...], 1, 1)
        info = _dot(q, rag_ref[bi], 1, 1)
        x1 = x1 + 0.1 * _dot(info, wrago_ref[...], 1, 1)
        t1 = _dot(x1, womin_ref[...], 1, 1)
        t2 = _dot(t1, qc, 1, 1)
        x1 = x1 + _dot(t2, womout_ref[...], 1, 1)
        logits = _dot(x1, wrt_ref[...], 1, 1)
        mx = jnp.max(logits, axis=-1, keepdims=True)
        ex = jnp.exp(logits - mx)
        probs = ex / jnp.sum(ex, axis=-1, keepdims=True)
        m1 = jnp.max(probs, axis=-1, keepdims=True)
        i1 = jnp.min(jnp.where(probs == m1, io, E), axis=-1, keepdims=True)
        mask1 = io == i1
        p2 = jnp.where(mask1, -1.0, probs)
        m2 = jnp.max(p2, axis=-1, keepdims=True)
        i2 = jnp.min(jnp.where(p2 == m2, io, E), axis=-1, keepdims=True)
        maskf = jnp.logical_or(mask1, io == i2).astype(_F32)
        gate = probs * maskf
        gate = gate / (jnp.sum(gate, axis=-1, keepdims=True) + 1e-9)
        acc = jnp.zeros((Lz, D), _F32)
        for e in range(E):
            d_e = _dot(x1, la_ref[e], 1, 0)
            acc = acc + _dot(d_e * gate[:, e:e + 1], lb_ref[e], 1, 0)
        x2 = x1 + acc
        psum = jnp.sum(probs, axis=0, keepdims=True)
        msum = jnp.sum(maskf, axis=0, keepdims=True)
        auxb_ref[bi] = jnp.concatenate(
            [psum, msum, jnp.zeros((6, E), _F32)], axis=0)
        x = x_ref[bi]
        h_old = jnp.sum(x, axis=0, keepdims=True) * (1.0 / Lz)
        h_new = jnp.sum(x2, axis=0, keepdims=True) * (1.0 / Lz)
        nov = _sigmoid(_dot(h_old, wnova_ref[...], 1, 1)
                       + _dot(h_new, wnovb_ref[...], 1, 1) + bnov_ref[0])
        x3 = nov * x2 + (1.0 - nov) * x
        h_post = nov * h_new + (1.0 - nov) * h_old
        hq = _dot(h_post, wmq_ref[...], 1, 1)
        mem = mem_ref[bi]
        num = jnp.sum(hq * mem)
        den = (jnp.sqrt(jnp.sum(hq * hq)) * jnp.sqrt(jnp.sum(mem * mem))
               + 1e-8)
        sim = num / den
        gm = _sigmoid(_dot(h_post, wmga_ref[...], 1, 1)
                      + _dot(mem, wmgb_ref[...], 1, 1) + bmg_ref[0])
        x4_ref[bi] = x3 + (sim * gm) * _dot(mem, wmp_ref[...], 1, 1)


def _run_kb(x, x1, rag_state, mem2, W_ragq, W_rago, W_skew, W_om_in,
            W_om_out, W_router, lora_A, lora_B, Wnova, Wnovb, b_nov,
            W_mq, W_mp, Wmga, Wmgb, b_mg):
    in_specs = ([_VMEM] * 14 + [_SMEM] + [_VMEM] * 4 + [_SMEM])
    outs = (
        jax.ShapeDtypeStruct((Bz, Lz, D), _F32),
        jax.ShapeDtypeStruct((Bz, 8, E), _F32),
    )
    return pl.pallas_call(
        _kb_body,
        out_shape=outs,
        in_specs=in_specs,
        out_specs=(_VMEM, _VMEM),
        compiler_params=_cp(),
        name="tars_kb_tail",
    )(x, x1, rag_state, mem2, W_ragq, W_rago, W_skew, W_om_in, W_om_out,
      W_router, lora_A, lora_B, Wnova, Wnovb, b_nov, W_mq, W_mp, Wmga,
      Wmgb, b_mg)


def kernel(x, wkv_state, x_prev, memory_vec, rag_state, ssd_state,
           conv_state, norm_scale, norm_bias, W_in, conv_w, conv_b, W_dt,
           dt_bias, A_log, W_B, W_C, W_ssd_out, W_r, W_k, W_v, W_w,
           W_rwkv_out, W_skew, W_om_in, W_om_out, W_router, lora_A,
           lora_B, W_nov, b_nov, W_ragq, W_rago, W_mq, W_mp, W_mg, b_mg):
    f = _F32
    Wz = W_in[:D]
    Wu = W_in[D:]
    Wdbc = jnp.concatenate(
        [W_B, W_C, W_dt, jnp.zeros((256 - 2 * N - H, D), f)], axis=0)
    Wrkvw = jnp.concatenate([W_r, W_k, W_v, W_w], axis=0)
    dt_bias2 = dt_bias.reshape(1, H)
    stpad = jnp.pad(conv_state.transpose(0, 2, 1), ((0, 0), (4, 0), (0, 0)))
    cwt = jnp.pad(conv_w.T, ((0, 8 - KC), (0, 0)))
    conv_b2 = conv_b.reshape(1, D)
    alog2 = A_log.reshape(1, H)
    (x1, ssd_new, wkv_new, convt, xlast) = _run_ka(
        x, x_prev, norm_scale, norm_bias, Wz, Wu, Wdbc, Wrkvw, dt_bias2,
        stpad, cwt, conv_b2, alog2, ssd_state, wkv_state, W_ssd_out,
        W_rwkv_out)
    conv_new = convt[:, 0:4, :].transpose(0, 2, 1)

    Wnova = W_nov[:, :D]
    Wnovb = W_nov[:, D:]
    Wmga = W_mg[:, :D]
    Wmgb = W_mg[:, D:]
    x4, auxb = _run_kb(x, x1, rag_state, memory_vec.reshape(Bz, 1, MEMD),
                       W_ragq, W_rago, W_skew, W_om_in, W_om_out,
                       W_router, lora_A, lora_B, Wnova, Wnovb, b_nov,
                       W_mq, W_mp, Wmga, Wmgb, b_mg)

    pmean = jnp.sum(auxb[:, 0, :], axis=0) * (1.0 / (Bz * Lz))
    mmean = jnp.sum(auxb[:, 1, :], axis=0) * (1.0 / (Bz * Lz))
    aux = E * jnp.sum(pmean * mmean)
    return x4, wkv_new, xlast, ssd_new, conv_new, aux


# speedup vs baseline: 45.3108x; 1.0026x over previous
import jax
import jax.numpy as jnp
from jax import lax
from jax.experimental import pallas as pl
from jax.experimental.pallas import tpu as pltpu

D = 1024
N = 64
P = 64
H = 16
E = 8
R = 8
OM = 32
MEMD = 384
KC = 4
Bz, Lz = 2, 1024

CS = 128
NCS = Lz // CS
CR = 32
NCR = Lz // CR

_F32 = jnp.float32
_VMEM = pl.BlockSpec(memory_space=pltpu.VMEM)
_SMEM = pl.BlockSpec(memory_space=pltpu.SMEM)


def _cp():
    return pltpu.CompilerParams(vmem_limit_bytes=100 * 1024 * 1024)


def _dot(a, b, ca, cb):
    return lax.dot_general(a, b, (((ca,), (cb,)), ((), ())),
                           preferred_element_type=_F32)


def _ln(x, s, b):
    m = jnp.mean(x, axis=-1, keepdims=True)
    v = jnp.mean((x - m) ** 2, axis=-1, keepdims=True)
    return (x - m) * lax.rsqrt(v + 1e-5) * s + b


def _silu(x):
    return x * (1.0 / (1.0 + jnp.exp(-x)))


def _sigmoid(x):
    return 1.0 / (1.0 + jnp.exp(-x))


def _ka_body(x_ref, xp_ref, ns_ref, nb_ref, wz_ref, wu_ref, wdbc_ref,
             wrkvw_ref, dtb_ref, stpad_ref, cwt_ref, cb_ref, alog_ref,
             ssd0_ref, wkv0_ref, wssd_ref, wrwk_ref,
             x1_ref, ssdn_ref, wkvn_ref, convt_ref, xlast_ref,
             ue, uc, zbuf, ysbuf, sscr):
    s = ns_ref[...]
    bvec = nb_ref[...]
    A2 = -jnp.exp(alog_ref[...])
    it = lax.broadcasted_iota(jnp.int32, (CS, CS), 0)
    js = lax.broadcasted_iota(jnp.int32, (CS, CS), 1)
    itr = lax.broadcasted_iota(jnp.int32, (CR, CR), 0)
    jsr = lax.broadcasted_iota(jnp.int32, (CR, CR), 1)
    trilf = (it >= js).astype(_F32)
    trilr = (itr >= jsr).astype(_F32)
    mask3 = (lax.broadcasted_iota(jnp.int32, (CR, CR, 1), 0)
             >= lax.broadcasted_iota(jnp.int32, (CR, CR, 1), 1))
    neg = jnp.float32(-1e30)
    for bi in range(Bz):
        xn = _ln(x_ref[bi], s, bvec)
        xs = jnp.concatenate([xp_ref[bi], xn[:Lz - 1]], axis=0)
        mix = 0.5 * (xn + xs)
        zbuf[...] = _dot(xn, wz_ref[...], 1, 1)
        u = _dot(xn, wu_ref[...], 1, 1)
        pbc = _dot(xn, wdbc_ref[...], 1, 1)
        bm = pbc[:, 0:64]
        cm = pbc[:, 64:128]
        dtp = jax.nn.softplus(pbc[:, 128:144] + dtb_ref[...])
        prk = _dot(mix, wrkvw_ref[...], 1, 1)
        xlast_ref[bi] = xn[Lz - 1:Lz, :]

        ue[0:8, :] = stpad_ref[bi]
        ue[8:8 + Lz, :] = u
        convt_ref[bi, 0:4, :] = u[Lz - 4:Lz, :]
        convt_ref[bi, 4:8, :] = jnp.zeros((4, D), _F32)
        acc = cb_ref[...]
        for k in range(KC):
            acc = acc + ue[4 + k:4 + k + Lz, :] * cwt_ref[k:k + 1, :]
        uc[...] = _silu(acc)

        sscr[...] = ssd0_ref[bi]
        for c in range(NCS):
            r0 = c * CS
            dtc = dtp[r0:r0 + CS, :]
            bmc = bm[r0:r0 + CS, :]
            cmc = cm[r0:r0 + CS, :]
            cum = _dot(trilf, dtc, 1, 0)
            cumT = cum.T
            dtT = dtc.T
            tot = cum[CS - 1:CS, :]
            e0 = jnp.exp(cum * A2)
            w0 = jnp.exp((tot - cum) * A2) * dtc
            dvec = jnp.exp(tot * A2)
            cb = _dot(cmc, bmc, 1, 1)
            ys_parts = []
            s_new = []
            for h in range(H):
                ah = A2[:, h:h + 1]
                ccol = cum[:, h:h + 1]
                crow = cumT[h:h + 1, :]
                g = jnp.exp(jnp.where(it >= js, (ccol - crow) * ah, neg))
                g = g * dtT[h:h + 1, :]
                uch = uc[r0:r0 + CS, h * P:(h + 1) * P]
                sh = sscr[h]
                y = _dot(cb * g, uch, 1, 0)
                y = y + _dot(cmc, sh, 1, 1) * e0[:, h:h + 1]
                ys_parts.append(y)
                m = _dot(uch * w0[:, h:h + 1], bmc, 0, 0)
                s_new.append(sh * dvec[:, h:h + 1] + m)
            ysbuf[r0:r0 + CS, :] = jnp.concatenate(ys_parts, axis=1)
            sscr[...] = jnp.stack(s_new, axis=0)
        ssdn_ref[bi] = sscr[...]

        rfull = prk[:, 0:64]
        kfull = prk[:, 64:128]
        vfull = prk[:, 128:192]
        logw = -jnp.exp(prk[:, 192:256])
        swk = wkv0_ref[bi]
        yr_parts = []
        for c in range(NCR):
            r0 = c * CR
            lw = logw[r0:r0 + CR, :]
            el = _dot(trilr, lw, 1, 0)
            rc = rfull[r0:r0 + CR, :]
            kc = kfull[r0:r0 + CR, :]
            vc = vfull[r0:r0 + CR, :]
            kr = _dot(rc, kc, 1, 1)
            e0 = jnp.exp(el)
            diff = el[:, None, :] - el[None, :, :]
            pd = jnp.exp(jnp.where(mask3, diff, neg))
            x3 = pd * kr[:, :, None] * vc[None, :, :]
            y = jnp.sum(x3, axis=1)
            yr_parts.append(y + _dot(rc, swk, 1, 1) * e0)
            wts = jnp.exp(el[CR - 1:CR, :] - el)
            mc = _dot(vc * wts, kc, 0, 0)
            dcol = e0[CR - 1:CR, :].T
            swk = swk * dcol + mc
        wkvn_ref[bi] = swk
        yrb = jnp.concatenate(yr_parts, axis=0)

        y_ssd = _dot(ysbuf[...], wssd_ref[...], 1, 1)
        y_r = _dot(yrb, wrwk_ref[...], 1, 1)
        x1_ref[bi] = x_ref[bi] + _silu(zbuf[...]) * y_ssd + y_r


def _run_ka(x, x_prev, norm_scale, norm_bias, Wz, Wu, Wdbc, Wrkvw,
            dt_bias2, stpad, cwt, conv_b2, alog2, ssd_state, wkv_state,
            W_ssd_out, W_rwkv_out):
    outs = (
        jax.ShapeDtypeStruct((Bz, Lz, D), _F32),
        jax.ShapeDtypeStruct((Bz, H, P, N), _F32),
        jax.ShapeDtypeStruct((Bz, N, N), _F32),
        jax.ShapeDtypeStruct((Bz, 8, D), _F32),
        jax.ShapeDtypeStruct((Bz, 1, D), _F32),
    )
    return pl.pallas_call(
        _ka_body,
        out_shape=outs,
        in_specs=[_VMEM] * 17,
        out_specs=tuple([_VMEM] * 5),
        scratch_shapes=[
            pltpu.VMEM((Lz + 8, D), _F32),
            pltpu.VMEM((Lz, D), _F32),
            pltpu.VMEM((Lz, D), _F32),
            pltpu.VMEM((Lz, D), _F32),
            pltpu.VMEM((H, P, N), _F32),
        ],
        compiler_params=_cp(),
        name="tars_ka_core",
    )(x, x_prev, norm_scale, norm_bias, Wz, Wu, Wdbc, Wrkvw, dt_bias2,
      stpad, cwt, conv_b2, alog2, ssd_state, wkv_state, W_ssd_out,
      W_rwkv_out)


def _kb_body(x_ref, x1_ref, rag_ref, mem_ref, wragq_ref, wrago_ref,
             wsk_ref, womin_ref, womout_ref, wrt_ref, la_ref, lb_ref,
             wnova_ref, wnovb_ref, bnov_ref, wmq_ref, wmp_ref, wmga_ref,
             wmgb_ref, bmg_ref,
             x4_ref, auxb_ref):
    wsk = wsk_ref[...]
    ask = 0.5 * (wsk - wsk.T)
    eye = (lax.broadcasted_iota(jnp.int32, (OM, OM), 0)
           == lax.broadcasted_iota(jnp.int32, (OM, OM), 1)).astype(_F32)
    m = eye - ask
    r1 = jnp.max(jnp.sum(jnp.abs(m), axis=0))
    rinf = jnp.max(jnp.sum(jnp.abs(m), axis=1))
    xinv = m.T * (1.0 / (r1 * rinf))
    for _ in range(20):
        xinv = _dot(xinv, 2.0 * eye - _dot(m, xinv, 1, 0), 1, 0)
    qc = _dot(xinv, eye + ask, 1, 0)
    io = lax.broadcasted_iota(jnp.int32, (Lz, E), 1)
    for bi in range(Bz):
        x1 = x1_ref[bi]
        xm = jnp.sum(x1, axis=0, keepdims=True) * (1.0 / Lz)
        q = _dot(xm, wragq_ref[...], 1, 1)
        info = _dot(q, rag_ref[bi], 1, 1)
        x1 = x1 + 0.1 * _dot(info, wrago_ref[...], 1, 1)
        t1 = _dot(x1, womin_ref[...], 1, 1)
        t2 = _dot(t1, qc, 1, 1)
        x1 = x1 + _dot(t2, womout_ref[...], 1, 1)
        logits = _dot(x1, wrt_ref[...], 1, 1)
        mx = jnp.max(logits, axis=-1, keepdims=True)
        ex = jnp.exp(logits - mx)
        probs = ex / jnp.sum(ex, axis=-1, keepdims=True)
        m1 = jnp.max(probs, axis=-1, keepdims=True)
        i1 = jnp.min(jnp.where(probs == m1, io, E), axis=-1, keepdims=True)
        mask1 = io == i1
        p2 = jnp.where(mask1, -1.0, probs)
        m2 = jnp.max(p2, axis=-1, keepdims=True)
        i2 = jnp.min(jnp.where(p2 == m2, io, E), axis=-1, keepdims=True)
        maskf = jnp.logical_or(mask1, io == i2).astype(_F32)
        gate = probs * maskf
        gate = gate / (jnp.sum(gate, axis=-1, keepdims=True) + 1e-9)
        acc = jnp.zeros((Lz, D), _F32)
        for e in range(E):
            d_e = _dot(x1, la_ref[e], 1, 0)
            acc = acc + _dot(d_e * gate[:, e:e + 1], lb_ref[e], 1, 0)
        x2 = x1 + acc
        psum = jnp.sum(probs, axis=0, keepdims=True)
        msum = jnp.sum(maskf, axis=0, keepdims=True)
        auxb_ref[bi] = jnp.concatenate(
            [psum, msum, jnp.zeros((6, E), _F32)], axis=0)
        x = x_ref[bi]
        h_old = jnp.sum(x, axis=0, keepdims=True) * (1.0 / Lz)
        h_new = jnp.sum(x2, axis=0, keepdims=True) * (1.0 / Lz)
        nov = _sigmoid(_dot(h_old, wnova_ref[...], 1, 1)
                       + _dot(h_new, wnovb_ref[...], 1, 1) + bnov_ref[0])
        x3 = nov * x2 + (1.0 - nov) * x
        h_post = nov * h_new + (1.0 - nov) * h_old
        hq = _dot(h_post, wmq_ref[...], 1, 1)
        mem = mem_ref[bi]
        num = jnp.sum(hq * mem)
        den = (jnp.sqrt(jnp.sum(hq * hq)) * jnp.sqrt(jnp.sum(mem * mem))
               + 1e-8)
        sim = num / den
        gm = _sigmoid(_dot(h_post, wmga_ref[...], 1, 1)
                      + _dot(mem, wmgb_ref[...], 1, 1) + bmg_ref[0])
        x4_ref[bi] = x3 + (sim * gm) * _dot(mem, wmp_ref[...], 1, 1)


def _run_kb(x, x1, rag_state, mem2, W_ragq, W_rago, W_skew, W_om_in,
            W_om_out, W_router, lora_A, lora_B, Wnova, Wnovb, b_nov,
            W_mq, W_mp, Wmga, Wmgb, b_mg):
    in_specs = ([_VMEM] * 14 + [_SMEM] + [_VMEM] * 4 + [_SMEM])
    outs = (
        jax.ShapeDtypeStruct((Bz, Lz, D), _F32),
        jax.ShapeDtypeStruct((Bz, 8, E), _F32),
    )
    return pl.pallas_call(
        _kb_body,
        out_shape=outs,
        in_specs=in_specs,
        out_specs=(_VMEM, _VMEM),
        compiler_params=_cp(),
        name="tars_kb_tail",
    )(x, x1, rag_state, mem2, W_ragq, W_rago, W_skew, W_om_in, W_om_out,
      W_router, lora_A, lora_B, Wnova, Wnovb, b_nov, W_mq, W_mp, Wmga,
      Wmgb, b_mg)


def kernel(x, wkv_state, x_prev, memory_vec, rag_state, ssd_state,
           conv_state, norm_scale, norm_bias, W_in, conv_w, conv_b, W_dt,
           dt_bias, A_log, W_B, W_C, W_ssd_out, W_r, W_k, W_v, W_w,
           W_rwkv_out, W_skew, W_om_in, W_om_out, W_router, lora_A,
           lora_B, W_nov, b_nov, W_ragq, W_rago, W_mq, W_mp, W_mg, b_mg):
    f = _F32
    Wz = W_in[:D]
    Wu = W_in[D:]
    Wdbc = jnp.concatenate(
        [W_B, W_C, W_dt, jnp.zeros((256 - 2 * N - H, D), f)], axis=0)
    Wrkvw = jnp.concatenate([W_r, W_k, W_v, W_w], axis=0)
    dt_bias2 = dt_bias.reshape(1, H)
    stpad = jnp.pad(conv_state.transpose(0, 2, 1), ((0, 0), (4, 0), (0, 0)))
    cwt = jnp.pad(conv_w.T, ((0, 8 - KC), (0, 0)))
    conv_b2 = conv_b.reshape(1, D)
    alog2 = A_log.reshape(1, H)
    (x1, ssd_new, wkv_new, convt, xlast) = _run_ka(
        x, x_prev, norm_scale, norm_bias, Wz, Wu, Wdbc, Wrkvw, dt_bias2,
        stpad, cwt, conv_b2, alog2, ssd_state, wkv_state, W_ssd_out,
        W_rwkv_out)
    conv_new = convt[:, 0:4, :].transpose(0, 2, 1)

    Wnova = W_nov[:, :D]
    Wnovb = W_nov[:, D:]
    Wmga = W_mg[:, :D]
    Wmgb = W_mg[:, D:]
    x4, auxb = _run_kb(x, x1, rag_state, memory_vec.reshape(Bz, 1, MEMD),
                       W_ragq, W_rago, W_skew, W_om_in, W_om_out,
                       W_router, lora_A, lora_B, Wnova, Wnovb, b_nov,
                       W_mq, W_mp, Wmga, Wmgb, b_mg)

    pmean = jnp.sum(auxb[:, 0, :], axis=0) * (1.0 / (Bz * Lz))
    mmean = jnp.sum(auxb[:, 1, :], axis=0) * (1.0 / (Bz * Lz))
    aux = E * jnp.sum(pmean * mmean)
    return x4, wkv_new, xlast, ssd_new, conv_new, aux


# speedup vs baseline: 45.7328x; 1.0093x over previous
import jax
import jax.numpy as jnp
from jax import lax
from jax.experimental import pallas as pl
from jax.experimental.pallas import tpu as pltpu

D = 1024
N = 64
P = 64
H = 16
E = 8
R = 8
OM = 32
MEMD = 384
KC = 4
Bz, Lz = 2, 1024

CS = 128
NCS = Lz // CS
CR = 32
NCR = Lz // CR

_F32 = jnp.float32
_VMEM = pl.BlockSpec(memory_space=pltpu.VMEM)
_SMEM = pl.BlockSpec(memory_space=pltpu.SMEM)


def _cp():
    return pltpu.CompilerParams(vmem_limit_bytes=100 * 1024 * 1024)


def _dot(a, b, ca, cb):
    return lax.dot_general(a, b, (((ca,), (cb,)), ((), ())),
                           preferred_element_type=_F32)


def _ln(x, s, b):
    m = jnp.mean(x, axis=-1, keepdims=True)
    v = jnp.mean((x - m) ** 2, axis=-1, keepdims=True)
    return (x - m) * lax.rsqrt(v + 1e-5) * s + b


def _silu(x):
    return x * (1.0 / (1.0 + jnp.exp(-x)))


def _sigmoid(x):
    return 1.0 / (1.0 + jnp.exp(-x))


def _ka_body(x_ref, xp_ref, ns_ref, nb_ref, wz_ref, wu_ref, wdbc_ref,
             wrkvw_ref, dtb_ref, stpad_ref, cwt_ref, cb_ref, alog_ref,
             ssd0_ref, wkv0_ref, wssd_ref, wrwk_ref,
             x1_ref, ssdn_ref, wkvn_ref, convt_ref, xlast_ref,
             ue, uc, zbuf, ysbuf, sscr):
    s = ns_ref[...]
    bvec = nb_ref[...]
    A2 = -jnp.exp(alog_ref[...])
    it = lax.broadcasted_iota(jnp.int32, (CS, CS), 0)
    js = lax.broadcasted_iota(jnp.int32, (CS, CS), 1)
    itr = lax.broadcasted_iota(jnp.int32, (CR, CR), 0)
    jsr = lax.broadcasted_iota(jnp.int32, (CR, CR), 1)
    trilf = (it >= js).astype(_F32)
    trilr = (itr >= jsr).astype(_F32)
    mask3 = (lax.broadcasted_iota(jnp.int32, (CR, CR, 1), 0)
             >= lax.broadcasted_iota(jnp.int32, (CR, CR, 1), 1))
    neg = jnp.float32(-1e30)
    for bi in range(Bz):
        xn = _ln(x_ref[bi], s, bvec)
        xs = jnp.concatenate([xp_ref[bi], xn[:Lz - 1]], axis=0)
        mix = 0.5 * (xn + xs)
        xn16 = xn.astype(jnp.bfloat16)
        zbuf[...] = _dot(xn16, wz_ref[...], 1, 1)
        u = _dot(xn16, wu_ref[...], 1, 1)
        pbc = _dot(xn, wdbc_ref[...], 1, 1)
        bm = pbc[:, 0:64]
        cm = pbc[:, 64:128]
        dtp = jax.nn.softplus(pbc[:, 128:144] + dtb_ref[...])
        prk = _dot(mix, wrkvw_ref[...], 1, 1)
        xlast_ref[bi] = xn[Lz - 1:Lz, :]

        ue[0:8, :] = stpad_ref[bi]
        ue[8:8 + Lz, :] = u
        convt_ref[bi, 0:4, :] = u[Lz - 4:Lz, :]
        convt_ref[bi, 4:8, :] = jnp.zeros((4, D), _F32)
        acc = cb_ref[...]
        for k in range(KC):
            acc = acc + ue[4 + k:4 + k + Lz, :] * cwt_ref[k:k + 1, :]
        uc[...] = _silu(acc)

        sscr[...] = ssd0_ref[bi]
        for c in range(NCS):
            r0 = c * CS
            dtc = dtp[r0:r0 + CS, :]
            bmc = bm[r0:r0 + CS, :]
            cmc = cm[r0:r0 + CS, :]
            cum = _dot(trilf, dtc, 1, 0)
            cumT = cum.T
            dtT = dtc.T
            tot = cum[CS - 1:CS, :]
            e0 = jnp.exp(cum * A2)
            w0 = jnp.exp((tot - cum) * A2) * dtc
            dvec = jnp.exp(tot * A2)
            cb = _dot(cmc, bmc, 1, 1)
            ys_parts = []
            s_new = []
            for h in range(H):
                ah = A2[:, h:h + 1]
                ccol = cum[:, h:h + 1]
                crow = cumT[h:h + 1, :]
                g = jnp.exp(jnp.where(it >= js, (ccol - crow) * ah, neg))
                g = g * dtT[h:h + 1, :]
                uch = uc[r0:r0 + CS, h * P:(h + 1) * P]
                sh = sscr[h]
                y = _dot(cb * g, uch, 1, 0)
                y = y + _dot(cmc, sh, 1, 1) * e0[:, h:h + 1]
                ys_parts.append(y)
                m = _dot(uch * w0[:, h:h + 1], bmc, 0, 0)
                s_new.append(sh * dvec[:, h:h + 1] + m)
            ysbuf[r0:r0 + CS, :] = jnp.concatenate(ys_parts, axis=1)
            sscr[...] = jnp.stack(s_new, axis=0)
        ssdn_ref[bi] = sscr[...]

        rfull = prk[:, 0:64]
        kfull = prk[:, 64:128]
        vfull = prk[:, 128:192]
        logw = -jnp.exp(prk[:, 192:256])
        swk = wkv0_ref[bi]
        yr_parts = []
        for c in range(NCR):
            r0 = c * CR
            lw = logw[r0:r0 + CR, :]
            el = _dot(trilr, lw, 1, 0)
            rc = rfull[r0:r0 + CR, :]
            kc = kfull[r0:r0 + CR, :]
            vc = vfull[r0:r0 + CR, :]
            kr = _dot(rc, kc, 1, 1)
            e0 = jnp.exp(el)
            diff = el[:, None, :] - el[None, :, :]
            pd = jnp.exp(jnp.where(mask3, diff, neg))
            x3 = pd * kr[:, :, None] * vc[None, :, :]
            y = jnp.sum(x3, axis=1)
            yr_parts.append(y + _dot(rc, swk, 1, 1) * e0)
            wts = jnp.exp(el[CR - 1:CR, :] - el)
            mc = _dot(vc * wts, kc, 0, 0)
            dcol = e0[CR - 1:CR, :].T
            swk = swk * dcol + mc
        wkvn_ref[bi] = swk
        yrb = jnp.concatenate(yr_parts, axis=0)

        y_ssd = _dot(ysbuf[...].astype(jnp.bfloat16), wssd_ref[...], 1, 1)
        y_r = _dot(yrb.astype(jnp.bfloat16), wrwk_ref[...], 1, 1)
        x1_ref[bi] = x_ref[bi] + _silu(zbuf[...]) * y_ssd + y_r


def _run_ka(x, x_prev, norm_scale, norm_bias, Wz, Wu, Wdbc, Wrkvw,
            dt_bias2, stpad, cwt, conv_b2, alog2, ssd_state, wkv_state,
            W_ssd_out, W_rwkv_out):
    outs = (
        jax.ShapeDtypeStruct((Bz, Lz, D), _F32),
        jax.ShapeDtypeStruct((Bz, H, P, N), _F32),
        jax.ShapeDtypeStruct((Bz, N, N), _F32),
        jax.ShapeDtypeStruct((Bz, 8, D), _F32),
        jax.ShapeDtypeStruct((Bz, 1, D), _F32),
    )
    return pl.pallas_call(
        _ka_body,
        out_shape=outs,
        in_specs=[_VMEM] * 17,
        out_specs=tuple([_VMEM] * 5),
        scratch_shapes=[
            pltpu.VMEM((Lz + 8, D), _F32),
            pltpu.VMEM((Lz, D), _F32),
            pltpu.VMEM((Lz, D), _F32),
            pltpu.VMEM((Lz, D), _F32),
            pltpu.VMEM((H, P, N), _F32),
        ],
        compiler_params=_cp(),
        name="tars_ka_core",
    )(x, x_prev, norm_scale, norm_bias, Wz, Wu, Wdbc, Wrkvw, dt_bias2,
      stpad, cwt, conv_b2, alog2, ssd_state, wkv_state, W_ssd_out,
      W_rwkv_out)


def _kb_body(x_ref, x1_ref, rag_ref, mem_ref, wragq_ref, wrago_ref,
             wsk_ref, womin_ref, womout_ref, wrt_ref, la_ref, lb_ref,
             wnova_ref, wnovb_ref, bnov_ref, wmq_ref, wmp_ref, wmga_ref,
             wmgb_ref, bmg_ref,
             x4_ref, auxb_ref):
    wsk = wsk_ref[...]
    ask = 0.5 * (wsk - wsk.T)
    eye = (lax.broadcasted_iota(jnp.int32, (OM, OM), 0)
           == lax.broadcasted_iota(jnp.int32, (OM, OM), 1)).astype(_F32)
    m = eye - ask
    r1 = jnp.max(jnp.sum(jnp.abs(m), axis=0))
    rinf = jnp.max(jnp.sum(jnp.abs(m), axis=1))
    xinv = m.T * (1.0 / (r1 * rinf))
    for _ in range(20):
        xinv = _dot(xinv, 2.0 * eye - _dot(m, xinv, 1, 0), 1, 0)
    qc = _dot(xinv, eye + ask, 1, 0)
    io = lax.broadcasted_iota(jnp.int32, (Lz, E), 1)
    for bi in range(Bz):
        x1 = x1_ref[bi]
        xm = jnp.sum(x1, axis=0, keepdims=True) * (1.0 / Lz)
        q = _dot(xm, wragq_ref[...], 1, 1)
        info = _dot(q, rag_ref[bi], 1, 1)
        x1 = x1 + 0.1 * _dot(info, wrago_ref[...], 1, 1)
        t1 = _dot(x1.astype(jnp.bfloat16), womin_ref[...], 1, 1)
        t2 = _dot(t1, qc, 1, 1)
        x1 = x1 + _dot(t2.astype(jnp.bfloat16), womout_ref[...], 1, 1)
        x116 = x1.astype(jnp.bfloat16)
        logits = _dot(x116, wrt_ref[...], 1, 1)
        mx = jnp.max(logits, axis=-1, keepdims=True)
        ex = jnp.exp(logits - mx)
        probs = ex / jnp.sum(ex, axis=-1, keepdims=True)
        m1 = jnp.max(probs, axis=-1, keepdims=True)
        i1 = jnp.min(jnp.where(probs == m1, io, E), axis=-1, keepdims=True)
        mask1 = io == i1
        p2 = jnp.where(mask1, -1.0, probs)
        m2 = jnp.max(p2, axis=-1, keepdims=True)
        i2 = jnp.min(jnp.where(p2 == m2, io, E), axis=-1, keepdims=True)
        maskf = jnp.logical_or(mask1, io == i2).astype(_F32)
        gate = probs * maskf
        gate = gate / (jnp.sum(gate, axis=-1, keepdims=True) + 1e-9)
        acc = jnp.zeros((Lz, D), _F32)
        for e in range(E):
            d_e = _dot(x116, la_ref[e], 1, 0)
            acc = acc + _dot((d_e * gate[:, e:e + 1]).astype(jnp.bfloat16),
                             lb_ref[e], 1, 0)
        x2 = x1 + acc
        psum = jnp.sum(probs, axis=0, keepdims=True)
        msum = jnp.sum(maskf, axis=0, keepdims=True)
        auxb_ref[bi] = jnp.concatenate(
            [psum, msum, jnp.zeros((6, E), _F32)], axis=0)
        x = x_ref[bi]
        h_old = jnp.sum(x, axis=0, keepdims=True) * (1.0 / Lz)
        h_new = jnp.sum(x2, axis=0, keepdims=True) * (1.0 / Lz)
        nov = _sigmoid(_dot(h_old, wnova_ref[...], 1, 1)
                       + _dot(h_new, wnovb_ref[...], 1, 1) + bnov_ref[0])
        x3 = nov * x2 + (1.0 - nov) * x
        h_post = nov * h_new + (1.0 - nov) * h_old
        hq = _dot(h_post, wmq_ref[...], 1, 1)
        mem = mem_ref[bi]
        num = jnp.sum(hq * mem)
        den = (jnp.sqrt(jnp.sum(hq * hq)) * jnp.sqrt(jnp.sum(mem * mem))
               + 1e-8)
        sim = num / den
        gm = _sigmoid(_dot(h_post, wmga_ref[...], 1, 1)
                      + _dot(mem, wmgb_ref[...], 1, 1) + bmg_ref[0])
        x4_ref[bi] = x3 + (sim * gm) * _dot(mem, wmp_ref[...], 1, 1)


def _run_kb(x, x1, rag_state, mem2, W_ragq, W_rago, W_skew, W_om_in,
            W_om_out, W_router, lora_A, lora_B, Wnova, Wnovb, b_nov,
            W_mq, W_mp, Wmga, Wmgb, b_mg):
    in_specs = ([_VMEM] * 14 + [_SMEM] + [_VMEM] * 4 + [_SMEM])
    outs = (
        jax.ShapeDtypeStruct((Bz, Lz, D), _F32),
        jax.ShapeDtypeStruct((Bz, 8, E), _F32),
    )
    return pl.pallas_call(
        _kb_body,
        out_shape=outs,
        in_specs=in_specs,
        out_specs=(_VMEM, _VMEM),
        compiler_params=_cp(),
        name="tars_kb_tail",
    )(x, x1, rag_state, mem2, W_ragq, W_rago, W_skew, W_om_in, W_om_out,
      W_router, lora_A, lora_B, Wnova, Wnovb, b_nov, W_mq, W_mp, Wmga,
      Wmgb, b_mg)


def kernel(x, wkv_state, x_prev, memory_vec, rag_state, ssd_state,
           conv_state, norm_scale, norm_bias, W_in, conv_w, conv_b, W_dt,
           dt_bias, A_log, W_B, W_C, W_ssd_out, W_r, W_k, W_v, W_w,
           W_rwkv_out, W_skew, W_om_in, W_om_out, W_router, lora_A,
           lora_B, W_nov, b_nov, W_ragq, W_rago, W_mq, W_mp, W_mg, b_mg):
    f = _F32
    Wz = W_in[:D].astype(jnp.bfloat16)
    Wu = W_in[D:].astype(jnp.bfloat16)
    Wdbc = jnp.concatenate(
        [W_B, W_C, W_dt, jnp.zeros((256 - 2 * N - H, D), f)], axis=0)
    Wrkvw = jnp.concatenate([W_r, W_k, W_v, W_w], axis=0)
    dt_bias2 = dt_bias.reshape(1, H)
    stpad = jnp.pad(conv_state.transpose(0, 2, 1), ((0, 0), (4, 0), (0, 0)))
    cwt = jnp.pad(conv_w.T, ((0, 8 - KC), (0, 0)))
    conv_b2 = conv_b.reshape(1, D)
    alog2 = A_log.reshape(1, H)
    (x1, ssd_new, wkv_new, convt, xlast) = _run_ka(
        x, x_prev, norm_scale, norm_bias, Wz, Wu, Wdbc, Wrkvw, dt_bias2,
        stpad, cwt, conv_b2, alog2, ssd_state, wkv_state,
        W_ssd_out.astype(jnp.bfloat16), W_rwkv_out.astype(jnp.bfloat16))
    conv_new = convt[:, 0:4, :].transpose(0, 2, 1)

    Wnova = W_nov[:, :D]
    Wnovb = W_nov[:, D:]
    Wmga = W_mg[:, :D]
    Wmgb = W_mg[:, D:]
    bf = jnp.bfloat16
    x4, auxb = _run_kb(x, x1, rag_state, memory_vec.reshape(Bz, 1, MEMD),
                       W_ragq, W_rago, W_skew, W_om_in.astype(bf),
                       W_om_out.astype(bf), W_router.astype(bf),
                       lora_A.astype(bf), lora_B.astype(bf), Wnova, Wnovb,
                       b_nov, W_mq, W_mp, Wmga, Wmgb, b_mg)

    pmean = jnp.sum(auxb[:, 0, :], axis=0) * (1.0 / (Bz * Lz))
    mmean = jnp.sum(auxb[:, 1, :], axis=0) * (1.0 / (Bz * Lz))
    aux = E * jnp.sum(pmean * mmean)
    return x4, wkv_new, xlast, ssd_new, conv_new, aux
